# baseline (speedup 1.0000x reference)
import functools

import jax
import jax.numpy as jnp
from jax import lax
from jax.experimental import pallas as pl
from jax.experimental.pallas import tpu as pltpu
from jax.experimental.pallas import tpu_sc as plsc

N_NODES = 10000
N_EDGES = 160000
EMB = 128
RBF_DIM = 510
RBF_PAD = 512
CUTOFF = 51.0
INV_GAP = 10.0
LN2 = 0.6931471805599453

NC = 2
NS = 16
NW = NC * NS
EDGES_PER_TILE = N_EDGES // NW
CHUNK = 64
NCH = 78
TAIL = EDGES_PER_TILE - NCH * CHUNK
NR = 2
NH = 3
STEADY0 = 2
NSTEADY = 12
PEEL_HI = 4

ROWS_A = 624
ROWS_LAST = N_NODES - ROWS_A * (NS - 1)


def _ssp(x):
    return jnp.maximum(x, 0.0) + jnp.log(1.0 + jnp.exp(-jnp.abs(x))) - LN2


def _hv0_body(nt_ref, embed_ref, w_ref, b_ref, out_ref):
    nt = nt_ref[...]
    e0 = embed_ref[0, :][None, :]
    e1 = embed_ref[1, :][None, :]
    h0 = jnp.where(nt > 0, e1, e0)
    out_ref[...] = (
        jnp.dot(h0, w_ref[...], preferred_element_type=jnp.float32) + b_ref[...]
    )


NCBLK = RBF_PAD // 128
GAPC = CUTOFF / (RBF_DIM - 1)
RBF_REACH = 2.0


def _edge_body(ef_ref, w1_ref, b1_ref, w2_ref, b2_ref, he0_ref, he1_ref,
               acc0_ref, acc1_ref):
    ef = ef_ref[...]
    d = jnp.sqrt(jnp.sum(ef * ef, axis=1, keepdims=True))
    acc0_ref[...] = jnp.zeros_like(acc0_ref)
    acc1_ref[...] = jnp.zeros_like(acc1_ref)
    cix = lax.broadcasted_iota(jnp.int32, (1, 128), 1).astype(jnp.float32)
    for j in range(NCBLK):
        lo = j * 128 * GAPC
        hi = (j * 128 + 127) * GAPC
        need = jnp.any((d >= lo - RBF_REACH) & (d <= hi + RBF_REACH))

        @pl.when(need)
        def _(j=j, lo=lo):
            centers = cix * GAPC + lo
            rbf = jnp.exp(-INV_GAP * (d - centers) ** 2).astype(jnp.bfloat16)
            acc0_ref[...] += jnp.dot(rbf, w1_ref[0, j],
                                     preferred_element_type=jnp.float32)
            acc1_ref[...] += jnp.dot(rbf, w1_ref[1, j],
                                     preferred_element_type=jnp.float32)

    for i, (acc_ref, out_ref) in enumerate(((acc0_ref, he0_ref),
                                            (acc1_ref, he1_ref))):
        t = _ssp(acc_ref[...] + b1_ref[i]).astype(jnp.bfloat16)
        t = _ssp(
            jnp.dot(t, w2_ref[i], preferred_element_type=jnp.float32) + b2_ref[i]
        )
        out_ref[...] = t


def _mid_body(p0_ref, p1_ref, cow_ref, cob_ref, pow_ref, pob_ref, pnw_ref,
              pnb_ref, out_ref):
    agg = p0_ref[...] + p1_ref[...]
    t = _ssp(jnp.dot(agg, cow_ref[...], preferred_element_type=jnp.float32)
             + cob_ref[...])
    h = jnp.dot(t, pow_ref[...], preferred_element_type=jnp.float32) + pob_ref[...]
    out_ref[...] = (
        jnp.dot(h, pnw_ref[...], preferred_element_type=jnp.float32) + pnb_ref[...]
    )


def _dec_body(p0_ref, p1_ref, cow_ref, cob_ref, pow_ref, pob_ref, decw_ref,
              decb_ref, pa_ref, out_ref):
    agg = p0_ref[...] + p1_ref[...]
    t = _ssp(jnp.dot(agg, cow_ref[...], preferred_element_type=jnp.float32)
             + cob_ref[...])
    x = jnp.dot(t, pow_ref[...], preferred_element_type=jnp.float32) + pob_ref[...]
    for j in range(4):
        x = jnp.dot(x, decw_ref[j], preferred_element_type=jnp.float32) + decb_ref[j]
        x = jnp.where(x >= 0, x, pa_ref[j] * x)
    out_ref[...] = (
        jnp.dot(x, decw_ref[4], preferred_element_type=jnp.float32) + decb_ref[4]
    )


def _full_spec(a):
    nd = a.ndim
    return pl.BlockSpec(a.shape, lambda i, _nd=nd: (0,) * _nd)


def _hv0(nfeats, embed, w, b):
    bn = 2000
    grid = (N_NODES // bn,)
    return pl.pallas_call(
        _hv0_body,
        grid=grid,
        in_specs=[
            pl.BlockSpec((bn, 1), lambda i: (i, 0)),
            _full_spec(embed),
            _full_spec(w),
            _full_spec(b),
        ],
        out_specs=pl.BlockSpec((bn, EMB), lambda i: (i, 0)),
        out_shape=jax.ShapeDtypeStruct((N_NODES, EMB), jnp.float32),
    )(nfeats, embed, w, b)


def _edge_mlp(efeats, w1p, b1, w2, b2):
    be = 640
    grid = (N_EDGES // be,)
    return pl.pallas_call(
        _edge_body,
        grid=grid,
        in_specs=[
            pl.BlockSpec((be, 16), lambda i: (i, 0)),
            _full_spec(w1p),
            _full_spec(b1),
            _full_spec(w2),
            _full_spec(b2),
        ],
        out_specs=[
            pl.BlockSpec((be, EMB), lambda i: (i, 0)),
            pl.BlockSpec((be, EMB), lambda i: (i, 0)),
        ],
        out_shape=[
            jax.ShapeDtypeStruct((N_EDGES, EMB), jnp.float32),
            jax.ShapeDtypeStruct((N_EDGES, EMB), jnp.float32),
        ],
        scratch_shapes=[
            pltpu.VMEM((be, EMB), jnp.float32),
            pltpu.VMEM((be, EMB), jnp.float32),
        ],
    )(efeats, w1p, b1, w2, b2)


def _mid(p0, p1, cow, cob, pow_, pob, pnw, pnb):
    bn = 2000
    grid = (N_NODES // bn,)
    return pl.pallas_call(
        _mid_body,
        grid=grid,
        in_specs=[
            pl.BlockSpec((bn, EMB), lambda i: (i, 0)),
            pl.BlockSpec((bn, EMB), lambda i: (i, 0)),
            _full_spec(cow), _full_spec(cob),
            _full_spec(pow_), _full_spec(pob),
            _full_spec(pnw), _full_spec(pnb),
        ],
        out_specs=pl.BlockSpec((bn, EMB), lambda i: (i, 0)),
        out_shape=jax.ShapeDtypeStruct((N_NODES, EMB), jnp.float32),
    )(p0, p1, cow, cob, pow_, pob, pnw, pnb)


def _decode(p0, p1, cow, cob, pow_, pob, decw, decb, pa):
    bn = 2000
    grid = (N_NODES // bn,)
    return pl.pallas_call(
        _dec_body,
        grid=grid,
        in_specs=[
            pl.BlockSpec((bn, EMB), lambda i: (i, 0)),
            pl.BlockSpec((bn, EMB), lambda i: (i, 0)),
            _full_spec(cow), _full_spec(cob),
            _full_spec(pow_), _full_spec(pob),
            _full_spec(decw), _full_spec(decb), _full_spec(pa),
        ],
        out_specs=pl.BlockSpec((bn, EMB), lambda i: (i, 0)),
        out_shape=jax.ShapeDtypeStruct((N_NODES, EMB), jnp.float32),
    )(p0, p1, cow, cob, pow_, pob, decw, decb, pa)


def _sc_body(hv_hbm, he_hbm, src_hbm, dst3_hbm, dtail_hbm, out_hbm,
             srcv_all, dstv0, dstv1, dstv2, dstv_t,
             rows0, rows1, hev0, hev1, hev2, rows_t, hev_t, acc,
             g0, g1, h0, h1, h2, s0, s1, s2, d0, d1, d2, tg, th, ts):
    c = lax.axis_index("c")
    s = lax.axis_index("s")
    gtile = c * NS + s
    ebase = gtile * EDGES_PER_TILE

    rows = (rows0, rows1)
    hevs = (hev0, hev1, hev2)
    dstv = (dstv0, dstv1, dstv2)
    gsem = (g0, g1)
    hsem = (h0, h1, h2)
    ssem = (s0, s1, s2)
    dsem = (d0, d1, d2)

    pltpu.sync_copy(src_hbm.at[pl.ds(ebase, EDGES_PER_TILE)], srcv_all)
    pltpu.sync_copy(dtail_hbm.at[gtile], dstv_t)

    def issue_gather(k, br):
        pltpu.async_copy(hv_hbm.at[srcv_all.at[pl.ds(k * CHUNK, CHUNK)]],
                         rows[br], gsem[br])

    def issue_he(k, bh):
        pltpu.async_copy(he_hbm.at[pl.ds(ebase + k * CHUNK, CHUNK)],
                         hevs[bh], hsem[bh])
        pltpu.async_copy(dst3_hbm.at[gtile, k], dstv[bh], dsem[bh])

    def wait_gather(br):
        pltpu.make_async_copy(hv_hbm.at[srcv_all.at[pl.ds(0, CHUNK)]],
                              rows[br], gsem[br]).wait()

    def wait_he(bh):
        pltpu.make_async_copy(he_hbm.at[pl.ds(ebase, CHUNK)],
                              hevs[bh], hsem[bh]).wait()

    def wait_dst(bh):
        pltpu.make_async_copy(dst3_hbm.at[gtile, 0], dstv[bh],
                              dsem[bh]).wait()

    def wait_scatter(bh):
        pltpu.make_async_copy(hevs[bh], acc.at[dstv[bh]], ssem[bh]).wait()

    def compute_scatter(k, br, bh):
        wait_gather(br)
        wait_he(bh)

        def mul(i, carry):
            for j in range(8):
                sl = pl.ds(j * 16, 16)
                hevs[bh][i, sl] = hevs[bh][i, sl] * rows[br][i, sl]
            return carry

        lax.fori_loop(0, CHUNK, mul, 0)
        wait_dst(bh)
        pltpu.async_copy(hevs[bh], acc.at[dstv[bh]], ssem[bh], add=True)

    def iteration(k, br, bh, bh_next, wait_sc):
        issue_gather(k + 1, 1 - br)
        if wait_sc:
            wait_scatter(bh_next)
        issue_he(k + 1, bh_next)
        compute_scatter(k, br, bh)

    issue_gather(0, 0)
    issue_he(0, 0)
    pltpu.async_copy(hv_hbm.at[srcv_all.at[pl.ds(NCH * CHUNK, TAIL)]],
                     rows_t, tg)
    pltpu.async_copy(he_hbm.at[pl.ds(ebase + NCH * CHUNK, TAIL)], hev_t, th)

    zero = jnp.zeros((16,), jnp.float32)

    def zbody(i, carry):
        for j in range(8):
            hev2[i, pl.ds(j * 16, 16)] = zero
        return carry

    lax.fori_loop(0, CHUNK, zbody, 0)
    base_r = s * ROWS_A

    @pl.when(s < NS - 1)
    def _():
        for k in range(9):
            pltpu.sync_copy(hev2, acc.at[pl.ds(base_r + k * CHUNK, CHUNK)])
        pltpu.sync_copy(hev2.at[pl.ds(0, ROWS_A - 9 * CHUNK)],
                        acc.at[pl.ds(base_r + 9 * CHUNK, ROWS_A - 9 * CHUNK)])

    @pl.when(s == NS - 1)
    def _():
        for k in range(10):
            pltpu.sync_copy(hev2, acc.at[pl.ds(base_r + k * CHUNK, CHUNK)])

    plsc.subcore_barrier()

    iteration(0, 0, 0, 1, False)
    iteration(1, 1, 1, 2, False)

    def steady(t, carry):
        k0 = STEADY0 + t * 6
        for d in range(6):
            k = k0 + d
            iteration(k, (STEADY0 + d) % NR, (STEADY0 + d) % NH,
                      (STEADY0 + d + 1) % NH, True)
        return carry

    lax.fori_loop(0, NSTEADY, steady, 0)
    k_hi = STEADY0 + NSTEADY * 6
    for k in range(k_hi, NCH - 1):
        iteration(k, k % NR, k % NH, (k + 1) % NH, True)
    compute_scatter(NCH - 1, (NCH - 1) % NR, (NCH - 1) % NH)
    for b in range(NH):
        wait_scatter(b)

    pltpu.make_async_copy(hv_hbm.at[srcv_all.at[pl.ds(NCH * CHUNK, TAIL)]],
                          rows_t, tg).wait()
    pltpu.make_async_copy(he_hbm.at[pl.ds(ebase, TAIL)], hev_t, th).wait()

    def mul_t(i, carry):
        for j in range(8):
            sl = pl.ds(j * 16, 16)
            hev_t[i, sl] = hev_t[i, sl] * rows_t[i, sl]
        return carry

    lax.fori_loop(0, TAIL, mul_t, 0)
    pltpu.async_copy(hev_t, acc.at[dstv_t], ts, add=True)
    pltpu.make_async_copy(hev_t, acc.at[dstv_t], ts).wait()

    plsc.subcore_barrier()

    @pl.when(s < NS - 1)
    def _():
        pltpu.sync_copy(acc.at[pl.ds(base_r, ROWS_A)],
                        out_hbm.at[c, pl.ds(base_r, ROWS_A)])

    @pl.when(s == NS - 1)
    def _():
        pltpu.sync_copy(acc.at[pl.ds(base_r, ROWS_LAST)],
                        out_hbm.at[c, pl.ds(base_r, ROWS_LAST)])


@functools.lru_cache(maxsize=1)
def _sc_gather_scatter_fn():
    return pl.kernel(
        _sc_body,
        out_type=jax.ShapeDtypeStruct((NC, N_NODES, EMB), jnp.float32),
        mesh=plsc.VectorSubcoreMesh(core_axis_name="c", subcore_axis_name="s",
                                    num_cores=NC, num_subcores=NS),
        scratch_types=[
            pltpu.VMEM((EDGES_PER_TILE,), jnp.int32),
            pltpu.VMEM((CHUNK,), jnp.int32),
            pltpu.VMEM((CHUNK,), jnp.int32),
            pltpu.VMEM((CHUNK,), jnp.int32),
            pltpu.VMEM((TAIL,), jnp.int32),
            pltpu.VMEM((CHUNK, EMB), jnp.float32),
            pltpu.VMEM((CHUNK, EMB), jnp.float32),
            pltpu.VMEM((CHUNK, EMB), jnp.float32),
            pltpu.VMEM((CHUNK, EMB), jnp.float32),
            pltpu.VMEM((CHUNK, EMB), jnp.float32),
            pltpu.VMEM((TAIL, EMB), jnp.float32),
            pltpu.VMEM((TAIL, EMB), jnp.float32),
            pltpu.VMEM_SHARED((N_NODES, EMB), jnp.float32),
        ] + [pltpu.SemaphoreType.DMA] * 14,
    )


def _sc_gather_scatter(hv, he, src, dst3, dtail):
    return _sc_gather_scatter_fn()(hv, he, src, dst3, dtail)


def kernel(nfeats, edge_index, efeats, embed, pn_W, pn_b, pe_W1, pe_b1, pe_W2,
           pe_b2, co_W, co_b, po_W, po_b, dec_W, dec_b, prelu_a):
    src = edge_index[0].astype(jnp.int32)
    dst = edge_index[1].astype(jnp.int32)
    nfeats = nfeats.astype(jnp.int32)
    dst_r = dst.reshape(NW, EDGES_PER_TILE)
    dst3 = dst_r[:, : NCH * CHUNK].reshape(NW, NCH, CHUNK)
    dtail = dst_r[:, NCH * CHUNK :]

    w1p = jnp.zeros((2, RBF_PAD, EMB), jnp.float32).at[:, :RBF_DIM, :].set(pe_W1)
    w1b = w1p.astype(jnp.bfloat16).reshape(2, NCBLK, 128, EMB)
    w2b = pe_W2.astype(jnp.bfloat16)
    pa = jnp.broadcast_to(prelu_a[:, None], (4, EMB))

    he0, he1 = _edge_mlp(efeats, w1b, pe_b1, w2b, pe_b2)
    hv0 = _hv0(nfeats, embed, pn_W[0], pn_b[0][None, :])

    parts0 = _sc_gather_scatter(hv0, he0, src, dst3, dtail)
    hv1 = _mid(parts0[0], parts0[1], co_W[0], co_b[0][None, :], po_W[0],
               po_b[0][None, :], pn_W[1], pn_b[1][None, :])

    parts1 = _sc_gather_scatter(hv1, he1, src, dst3, dtail)
    out = _decode(parts1[0], parts1[1], co_W[1], co_b[1][None, :], po_W[1],
                  po_b[1][None, :], dec_W, dec_b, pa)
    return out


# speedup vs baseline: 1.0605x; 1.0605x over previous
import functools

import jax
import jax.numpy as jnp
from jax import lax
from jax.experimental import pallas as pl
from jax.experimental.pallas import tpu as pltpu
from jax.experimental.pallas import tpu_sc as plsc

N_NODES = 10000
N_EDGES = 160000
EMB = 128
RBF_DIM = 510
RBF_PAD = 512
CUTOFF = 51.0
INV_GAP = 10.0
LN2 = 0.6931471805599453

NC = 2
NS = 16
NW = NC * NS
EDGES_PER_TILE = N_EDGES // NW
CHUNK = 64
NCH = 78
TAIL = EDGES_PER_TILE - NCH * CHUNK
NR = 2
NH = 3
STEADY0 = 2
NSTEADY = 12
PEEL_HI = 4

ROWS_A = 624
ROWS_LAST = N_NODES - ROWS_A * (NS - 1)


def _ssp(x):
    return jnp.maximum(x, 0.0) + jnp.log(1.0 + jnp.exp(-jnp.abs(x))) - LN2


def _hv0_body(nt_ref, embed_ref, w_ref, b_ref, out_ref):
    nt = nt_ref[...]
    e0 = embed_ref[0, :][None, :]
    e1 = embed_ref[1, :][None, :]
    h0 = jnp.where(nt > 0, e1, e0)
    out_ref[...] = (
        jnp.dot(h0, w_ref[...], preferred_element_type=jnp.float32) + b_ref[...]
    )


GAPC = CUTOFF / (RBF_DIM - 1)


def _edge_body(ef_ref, w1_ref, b1_ref, w2_ref, b2_ref, he0_ref, he1_ref):
    ef = ef_ref[...]
    d = jnp.sqrt(jnp.sum(ef * ef, axis=1, keepdims=True))
    centers = lax.broadcasted_iota(jnp.int32, (1, RBF_PAD), 1).astype(
        jnp.float32
    ) * GAPC
    rbf = jnp.exp(-INV_GAP * (d - centers) ** 2).astype(jnp.bfloat16)
    for i, out_ref in enumerate((he0_ref, he1_ref)):
        t = _ssp(
            jnp.dot(rbf, w1_ref[i], preferred_element_type=jnp.float32)
            + b1_ref[i]
        ).astype(jnp.bfloat16)
        t = _ssp(
            jnp.dot(t, w2_ref[i], preferred_element_type=jnp.float32) + b2_ref[i]
        )
        out_ref[...] = t


def _mid_body(p0_ref, p1_ref, cow_ref, cob_ref, pow_ref, pob_ref, pnw_ref,
              pnb_ref, out_ref):
    agg = p0_ref[...] + p1_ref[...]
    t = _ssp(jnp.dot(agg, cow_ref[...], preferred_element_type=jnp.float32)
             + cob_ref[...])
    h = jnp.dot(t, pow_ref[...], preferred_element_type=jnp.float32) + pob_ref[...]
    out_ref[...] = (
        jnp.dot(h, pnw_ref[...], preferred_element_type=jnp.float32) + pnb_ref[...]
    )


def _dec_body(p0_ref, p1_ref, cow_ref, cob_ref, pow_ref, pob_ref, decw_ref,
              decb_ref, pa_ref, out_ref):
    agg = p0_ref[...] + p1_ref[...]
    t = _ssp(jnp.dot(agg, cow_ref[...], preferred_element_type=jnp.float32)
             + cob_ref[...])
    x = jnp.dot(t, pow_ref[...], preferred_element_type=jnp.float32) + pob_ref[...]
    for j in range(4):
        x = jnp.dot(x, decw_ref[j], preferred_element_type=jnp.float32) + decb_ref[j]
        x = jnp.where(x >= 0, x, pa_ref[j] * x)
    out_ref[...] = (
        jnp.dot(x, decw_ref[4], preferred_element_type=jnp.float32) + decb_ref[4]
    )


def _full_spec(a):
    nd = a.ndim
    return pl.BlockSpec(a.shape, lambda i, _nd=nd: (0,) * _nd)


def _hv0(nfeats, embed, w, b):
    bn = 2000
    grid = (N_NODES // bn,)
    return pl.pallas_call(
        _hv0_body,
        grid=grid,
        in_specs=[
            pl.BlockSpec((bn, 1), lambda i: (i, 0)),
            _full_spec(embed),
            _full_spec(w),
            _full_spec(b),
        ],
        out_specs=pl.BlockSpec((bn, EMB), lambda i: (i, 0)),
        out_shape=jax.ShapeDtypeStruct((N_NODES, EMB), jnp.float32),
    )(nfeats, embed, w, b)


def _edge_mlp(efeats, w1p, b1, w2, b2):
    be = 640
    grid = (N_EDGES // be,)
    return pl.pallas_call(
        _edge_body,
        grid=grid,
        in_specs=[
            pl.BlockSpec((be, 16), lambda i: (i, 0)),
            _full_spec(w1p),
            _full_spec(b1),
            _full_spec(w2),
            _full_spec(b2),
        ],
        out_specs=[
            pl.BlockSpec((be, EMB), lambda i: (i, 0)),
            pl.BlockSpec((be, EMB), lambda i: (i, 0)),
        ],
        out_shape=[
            jax.ShapeDtypeStruct((N_EDGES, EMB), jnp.float32),
            jax.ShapeDtypeStruct((N_EDGES, EMB), jnp.float32),
        ],
    )(efeats, w1p, b1, w2, b2)


def _mid(p0, p1, cow, cob, pow_, pob, pnw, pnb):
    bn = 2000
    grid = (N_NODES // bn,)
    return pl.pallas_call(
        _mid_body,
        grid=grid,
        in_specs=[
            pl.BlockSpec((bn, EMB), lambda i: (i, 0)),
            pl.BlockSpec((bn, EMB), lambda i: (i, 0)),
            _full_spec(cow), _full_spec(cob),
            _full_spec(pow_), _full_spec(pob),
            _full_spec(pnw), _full_spec(pnb),
        ],
        out_specs=pl.BlockSpec((bn, EMB), lambda i: (i, 0)),
        out_shape=jax.ShapeDtypeStruct((N_NODES, EMB), jnp.float32),
    )(p0, p1, cow, cob, pow_, pob, pnw, pnb)


def _decode(p0, p1, cow, cob, pow_, pob, decw, decb, pa):
    bn = 2000
    grid = (N_NODES // bn,)
    return pl.pallas_call(
        _dec_body,
        grid=grid,
        in_specs=[
            pl.BlockSpec((bn, EMB), lambda i: (i, 0)),
            pl.BlockSpec((bn, EMB), lambda i: (i, 0)),
            _full_spec(cow), _full_spec(cob),
            _full_spec(pow_), _full_spec(pob),
            _full_spec(decw), _full_spec(decb), _full_spec(pa),
        ],
        out_specs=pl.BlockSpec((bn, EMB), lambda i: (i, 0)),
        out_shape=jax.ShapeDtypeStruct((N_NODES, EMB), jnp.float32),
    )(p0, p1, cow, cob, pow_, pob, decw, decb, pa)


def _sc_body(hv_hbm, he_hbm, src_hbm, dst3_hbm, dtail_hbm, out_hbm,
             srcv_all, dstv0, dstv1, dstv2, dstv_t,
             rows0, rows1, hev0, hev1, hev2, rows_t, hev_t, acc,
             g0, g1, h0, h1, h2, s0, s1, s2, d0, d1, d2, tg, th, ts):
    c = lax.axis_index("c")
    s = lax.axis_index("s")
    gtile = c * NS + s
    ebase = gtile * EDGES_PER_TILE

    rows = (rows0, rows1)
    hevs = (hev0, hev1, hev2)
    dstv = (dstv0, dstv1, dstv2)
    gsem = (g0, g1)
    hsem = (h0, h1, h2)
    ssem = (s0, s1, s2)
    dsem = (d0, d1, d2)

    pltpu.sync_copy(src_hbm.at[pl.ds(ebase, EDGES_PER_TILE)], srcv_all)
    pltpu.sync_copy(dtail_hbm.at[gtile], dstv_t)

    def issue_gather(k, br):
        pltpu.async_copy(hv_hbm.at[srcv_all.at[pl.ds(k * CHUNK, CHUNK)]],
                         rows[br], gsem[br])

    def issue_he(k, bh):
        pltpu.async_copy(he_hbm.at[pl.ds(ebase + k * CHUNK, CHUNK)],
                         hevs[bh], hsem[bh])
        pltpu.async_copy(dst3_hbm.at[gtile, k], dstv[bh], dsem[bh])

    def wait_gather(br):
        pltpu.make_async_copy(hv_hbm.at[srcv_all.at[pl.ds(0, CHUNK)]],
                              rows[br], gsem[br]).wait()

    def wait_he(bh):
        pltpu.make_async_copy(he_hbm.at[pl.ds(ebase, CHUNK)],
                              hevs[bh], hsem[bh]).wait()

    def wait_dst(bh):
        pltpu.make_async_copy(dst3_hbm.at[gtile, 0], dstv[bh],
                              dsem[bh]).wait()

    def wait_scatter(bh):
        pltpu.make_async_copy(hevs[bh], acc.at[dstv[bh]], ssem[bh]).wait()

    def compute_scatter(k, br, bh):
        wait_gather(br)
        wait_he(bh)

        def mul(i, carry):
            for j in range(8):
                sl = pl.ds(j * 16, 16)
                hevs[bh][i, sl] = hevs[bh][i, sl] * rows[br][i, sl]
            return carry

        lax.fori_loop(0, CHUNK, mul, 0)
        wait_dst(bh)
        pltpu.async_copy(hevs[bh], acc.at[dstv[bh]], ssem[bh], add=True)

    def iteration(k, br, bh, bh_next, wait_sc):
        issue_gather(k + 1, 1 - br)
        if wait_sc:
            wait_scatter(bh_next)
        issue_he(k + 1, bh_next)
        compute_scatter(k, br, bh)

    issue_gather(0, 0)
    issue_he(0, 0)
    pltpu.async_copy(hv_hbm.at[srcv_all.at[pl.ds(NCH * CHUNK, TAIL)]],
                     rows_t, tg)
    pltpu.async_copy(he_hbm.at[pl.ds(ebase + NCH * CHUNK, TAIL)], hev_t, th)

    zero = jnp.zeros((16,), jnp.float32)

    def zbody(i, carry):
        for j in range(8):
            hev2[i, pl.ds(j * 16, 16)] = zero
        return carry

    lax.fori_loop(0, CHUNK, zbody, 0)
    base_r = s * ROWS_A

    @pl.when(s < NS - 1)
    def _():
        for k in range(9):
            pltpu.sync_copy(hev2, acc.at[pl.ds(base_r + k * CHUNK, CHUNK)])
        pltpu.sync_copy(hev2.at[pl.ds(0, ROWS_A - 9 * CHUNK)],
                        acc.at[pl.ds(base_r + 9 * CHUNK, ROWS_A - 9 * CHUNK)])

    @pl.when(s == NS - 1)
    def _():
        for k in range(10):
            pltpu.sync_copy(hev2, acc.at[pl.ds(base_r + k * CHUNK, CHUNK)])

    plsc.subcore_barrier()

    iteration(0, 0, 0, 1, False)
    iteration(1, 1, 1, 2, False)

    def steady(t, carry):
        k0 = STEADY0 + t * 6
        for d in range(6):
            k = k0 + d
            iteration(k, (STEADY0 + d) % NR, (STEADY0 + d) % NH,
                      (STEADY0 + d + 1) % NH, True)
        return carry

    lax.fori_loop(0, NSTEADY, steady, 0)
    k_hi = STEADY0 + NSTEADY * 6
    for k in range(k_hi, NCH - 1):
        iteration(k, k % NR, k % NH, (k + 1) % NH, True)
    compute_scatter(NCH - 1, (NCH - 1) % NR, (NCH - 1) % NH)
    for b in range(NH):
        wait_scatter(b)

    pltpu.make_async_copy(hv_hbm.at[srcv_all.at[pl.ds(NCH * CHUNK, TAIL)]],
                          rows_t, tg).wait()
    pltpu.make_async_copy(he_hbm.at[pl.ds(ebase, TAIL)], hev_t, th).wait()

    def mul_t(i, carry):
        for j in range(8):
            sl = pl.ds(j * 16, 16)
            hev_t[i, sl] = hev_t[i, sl] * rows_t[i, sl]
        return carry

    lax.fori_loop(0, TAIL, mul_t, 0)
    pltpu.async_copy(hev_t, acc.at[dstv_t], ts, add=True)
    pltpu.make_async_copy(hev_t, acc.at[dstv_t], ts).wait()

    plsc.subcore_barrier()

    @pl.when(s < NS - 1)
    def _():
        pltpu.sync_copy(acc.at[pl.ds(base_r, ROWS_A)],
                        out_hbm.at[c, pl.ds(base_r, ROWS_A)])

    @pl.when(s == NS - 1)
    def _():
        pltpu.sync_copy(acc.at[pl.ds(base_r, ROWS_LAST)],
                        out_hbm.at[c, pl.ds(base_r, ROWS_LAST)])


@functools.lru_cache(maxsize=1)
def _sc_gather_scatter_fn():
    return pl.kernel(
        _sc_body,
        out_type=jax.ShapeDtypeStruct((NC, N_NODES, EMB), jnp.float32),
        mesh=plsc.VectorSubcoreMesh(core_axis_name="c", subcore_axis_name="s",
                                    num_cores=NC, num_subcores=NS),
        scratch_types=[
            pltpu.VMEM((EDGES_PER_TILE,), jnp.int32),
            pltpu.VMEM((CHUNK,), jnp.int32),
            pltpu.VMEM((CHUNK,), jnp.int32),
            pltpu.VMEM((CHUNK,), jnp.int32),
            pltpu.VMEM((TAIL,), jnp.int32),
            pltpu.VMEM((CHUNK, EMB), jnp.float32),
            pltpu.VMEM((CHUNK, EMB), jnp.float32),
            pltpu.VMEM((CHUNK, EMB), jnp.float32),
            pltpu.VMEM((CHUNK, EMB), jnp.float32),
            pltpu.VMEM((CHUNK, EMB), jnp.float32),
            pltpu.VMEM((TAIL, EMB), jnp.float32),
            pltpu.VMEM((TAIL, EMB), jnp.float32),
            pltpu.VMEM_SHARED((N_NODES, EMB), jnp.float32),
        ] + [pltpu.SemaphoreType.DMA] * 14,
    )


def _sc_gather_scatter(hv, he, src, dst3, dtail):
    return _sc_gather_scatter_fn()(hv, he, src, dst3, dtail)


def kernel(nfeats, edge_index, efeats, embed, pn_W, pn_b, pe_W1, pe_b1, pe_W2,
           pe_b2, co_W, co_b, po_W, po_b, dec_W, dec_b, prelu_a):
    src = edge_index[0].astype(jnp.int32)
    dst = edge_index[1].astype(jnp.int32)
    nfeats = nfeats.astype(jnp.int32)
    dst_r = dst.reshape(NW, EDGES_PER_TILE)
    dst3 = dst_r[:, : NCH * CHUNK].reshape(NW, NCH, CHUNK)
    dtail = dst_r[:, NCH * CHUNK :]

    w1p = jnp.zeros((2, RBF_PAD, EMB), jnp.float32).at[:, :RBF_DIM, :].set(pe_W1)
    w1b = w1p.astype(jnp.bfloat16)
    w2b = pe_W2.astype(jnp.bfloat16)
    pa = jnp.broadcast_to(prelu_a[:, None], (4, EMB))

    he0, he1 = _edge_mlp(efeats, w1b, pe_b1, w2b, pe_b2)
    hv0 = _hv0(nfeats, embed, pn_W[0], pn_b[0][None, :])

    parts0 = _sc_gather_scatter(hv0, he0, src, dst3, dtail)
    hv1 = _mid(parts0[0], parts0[1], co_W[0], co_b[0][None, :], po_W[0],
               po_b[0][None, :], pn_W[1], pn_b[1][None, :])

    parts1 = _sc_gather_scatter(hv1, he1, src, dst3, dtail)
    out = _decode(parts1[0], parts1[1], co_W[1], co_b[1][None, :], po_W[1],
                  po_b[1][None, :], dec_W, dec_b, pa)
    return out


# speedup vs baseline: 1.1416x; 1.0764x over previous
import functools

import jax
import jax.numpy as jnp
from jax import lax
from jax.experimental import pallas as pl
from jax.experimental.pallas import tpu as pltpu
from jax.experimental.pallas import tpu_sc as plsc

N_NODES = 10000
N_EDGES = 160000
EMB = 128
RBF_DIM = 510
RBF_PAD = 512
CUTOFF = 51.0
INV_GAP = 10.0
LN2 = 0.6931471805599453

NC = 2
NS = 16
NW = NC * NS
EDGES_PER_TILE = N_EDGES // NW
CHUNK = 64
NCH = 78
TAIL = EDGES_PER_TILE - NCH * CHUNK
NR = 2
NH = 3
STEADY0 = 2
NSTEADY = 12
PEEL_HI = 4

ROWS_A = 624
ROWS_LAST = N_NODES - ROWS_A * (NS - 1)


def _ssp(x):
    return jnp.maximum(x, 0.0) + jnp.log(1.0 + jnp.exp(-jnp.abs(x))) - LN2


def _hv0_body(nt_ref, embed_ref, w_ref, b_ref, out_ref):
    nt = nt_ref[...]
    e0 = embed_ref[0, :][None, :]
    e1 = embed_ref[1, :][None, :]
    h0 = jnp.where(nt > 0, e1, e0)
    out_ref[...] = (
        jnp.dot(h0, w_ref[...], preferred_element_type=jnp.float32) + b_ref[...]
    )


GAPC = CUTOFF / (RBF_DIM - 1)


def _edge_body(ef_ref, w1_ref, b1_ref, w2_ref, b2_ref, he0_ref, he1_ref):
    ef = ef_ref[...]
    sq = INV_GAP ** 0.5
    u = jnp.sqrt(jnp.sum(ef * ef, axis=1, keepdims=True)) * sq
    centers = lax.broadcasted_iota(jnp.int32, (1, RBF_PAD), 1).astype(
        jnp.float32
    ) * (GAPC * sq)
    rbf = jnp.exp(-((u - centers) ** 2)).astype(jnp.bfloat16)
    for i, out_ref in enumerate((he0_ref, he1_ref)):
        t = _ssp(
            jnp.dot(rbf, w1_ref[i], preferred_element_type=jnp.float32)
            + b1_ref[i]
        ).astype(jnp.bfloat16)
        t = _ssp(
            jnp.dot(t, w2_ref[i], preferred_element_type=jnp.float32) + b2_ref[i]
        )
        out_ref[...] = t


def _mid_body(p0_ref, p1_ref, cow_ref, cob_ref, pow_ref, pob_ref, pnw_ref,
              pnb_ref, out_ref):
    agg = p0_ref[...] + p1_ref[...]
    t = _ssp(jnp.dot(agg, cow_ref[...], preferred_element_type=jnp.float32)
             + cob_ref[...])
    h = jnp.dot(t, pow_ref[...], preferred_element_type=jnp.float32) + pob_ref[...]
    out_ref[...] = (
        jnp.dot(h, pnw_ref[...], preferred_element_type=jnp.float32) + pnb_ref[...]
    )


def _dec_body(p0_ref, p1_ref, cow_ref, cob_ref, pow_ref, pob_ref, decw_ref,
              decb_ref, pa_ref, out_ref):
    agg = p0_ref[...] + p1_ref[...]
    t = _ssp(jnp.dot(agg, cow_ref[...], preferred_element_type=jnp.float32)
             + cob_ref[...])
    x = jnp.dot(t, pow_ref[...], preferred_element_type=jnp.float32) + pob_ref[...]
    for j in range(4):
        x = jnp.dot(x, decw_ref[j], preferred_element_type=jnp.float32) + decb_ref[j]
        x = jnp.where(x >= 0, x, pa_ref[j] * x)
    out_ref[...] = (
        jnp.dot(x, decw_ref[4], preferred_element_type=jnp.float32) + decb_ref[4]
    )


def _full_spec(a):
    nd = a.ndim
    return pl.BlockSpec(a.shape, lambda i, _nd=nd: (0,) * _nd)


def _hv0(nfeats, embed, w, b):
    bn = 2000
    grid = (N_NODES // bn,)
    return pl.pallas_call(
        _hv0_body,
        grid=grid,
        in_specs=[
            pl.BlockSpec((bn, 1), lambda i: (i, 0)),
            _full_spec(embed),
            _full_spec(w),
            _full_spec(b),
        ],
        out_specs=pl.BlockSpec((bn, EMB), lambda i: (i, 0)),
        out_shape=jax.ShapeDtypeStruct((N_NODES, EMB), jnp.float32),
    )(nfeats, embed, w, b)


def _edge_mlp(efeats, w1p, b1, w2, b2):
    be = 1600
    grid = (N_EDGES // be,)
    return pl.pallas_call(
        _edge_body,
        grid=grid,
        in_specs=[
            pl.BlockSpec((be, 16), lambda i: (i, 0)),
            _full_spec(w1p),
            _full_spec(b1),
            _full_spec(w2),
            _full_spec(b2),
        ],
        out_specs=[
            pl.BlockSpec((be, EMB), lambda i: (i, 0)),
            pl.BlockSpec((be, EMB), lambda i: (i, 0)),
        ],
        out_shape=[
            jax.ShapeDtypeStruct((N_EDGES, EMB), jnp.float32),
            jax.ShapeDtypeStruct((N_EDGES, EMB), jnp.float32),
        ],
    )(efeats, w1p, b1, w2, b2)


def _mid(p0, p1, cow, cob, pow_, pob, pnw, pnb):
    bn = 2000
    grid = (N_NODES // bn,)
    return pl.pallas_call(
        _mid_body,
        grid=grid,
        in_specs=[
            pl.BlockSpec((bn, EMB), lambda i: (i, 0)),
            pl.BlockSpec((bn, EMB), lambda i: (i, 0)),
            _full_spec(cow), _full_spec(cob),
            _full_spec(pow_), _full_spec(pob),
            _full_spec(pnw), _full_spec(pnb),
        ],
        out_specs=pl.BlockSpec((bn, EMB), lambda i: (i, 0)),
        out_shape=jax.ShapeDtypeStruct((N_NODES, EMB), jnp.float32),
    )(p0, p1, cow, cob, pow_, pob, pnw, pnb)


def _decode(p0, p1, cow, cob, pow_, pob, decw, decb, pa):
    bn = 2000
    grid = (N_NODES // bn,)
    return pl.pallas_call(
        _dec_body,
        grid=grid,
        in_specs=[
            pl.BlockSpec((bn, EMB), lambda i: (i, 0)),
            pl.BlockSpec((bn, EMB), lambda i: (i, 0)),
            _full_spec(cow), _full_spec(cob),
            _full_spec(pow_), _full_spec(pob),
            _full_spec(decw), _full_spec(decb), _full_spec(pa),
        ],
        out_specs=pl.BlockSpec((bn, EMB), lambda i: (i, 0)),
        out_shape=jax.ShapeDtypeStruct((N_NODES, EMB), jnp.float32),
    )(p0, p1, cow, cob, pow_, pob, decw, decb, pa)


def _sc_body(hv_hbm, he_hbm, src_hbm, dst3_hbm, dtail_hbm, out_hbm,
             srcv_all, dstv0, dstv1, dstv2, dstv_t,
             rows0, rows1, hev0, hev1, hev2, rows_t, hev_t, acc,
             g0, g1, h0, h1, h2, s0, s1, s2, d0, d1, d2, tg, th, ts):
    c = lax.axis_index("c")
    s = lax.axis_index("s")
    gtile = c * NS + s
    ebase = gtile * EDGES_PER_TILE

    rows = (rows0, rows1)
    hevs = (hev0, hev1, hev2)
    dstv = (dstv0, dstv1, dstv2)
    gsem = (g0, g1)
    hsem = (h0, h1, h2)
    ssem = (s0, s1, s2)
    dsem = (d0, d1, d2)

    pltpu.sync_copy(src_hbm.at[pl.ds(ebase, EDGES_PER_TILE)], srcv_all)
    pltpu.sync_copy(dtail_hbm.at[gtile], dstv_t)

    def issue_gather(k, br):
        pltpu.async_copy(hv_hbm.at[srcv_all.at[pl.ds(k * CHUNK, CHUNK)]],
                         rows[br], gsem[br])

    def issue_he(k, bh):
        pltpu.async_copy(he_hbm.at[pl.ds(ebase + k * CHUNK, CHUNK)],
                         hevs[bh], hsem[bh])
        pltpu.async_copy(dst3_hbm.at[gtile, k], dstv[bh], dsem[bh])

    def wait_gather(br):
        pltpu.make_async_copy(hv_hbm.at[srcv_all.at[pl.ds(0, CHUNK)]],
                              rows[br], gsem[br]).wait()

    def wait_he(bh):
        pltpu.make_async_copy(he_hbm.at[pl.ds(ebase, CHUNK)],
                              hevs[bh], hsem[bh]).wait()

    def wait_dst(bh):
        pltpu.make_async_copy(dst3_hbm.at[gtile, 0], dstv[bh],
                              dsem[bh]).wait()

    def wait_scatter(bh):
        pltpu.make_async_copy(hevs[bh], acc.at[dstv[bh]], ssem[bh]).wait()

    def compute_scatter(k, br, bh):
        wait_gather(br)
        wait_he(bh)

        def mul(i4, carry):
            for r in range(4):
                i = i4 * 4 + r
                for j in range(8):
                    sl = pl.ds(j * 16, 16)
                    hevs[bh][i, sl] = hevs[bh][i, sl] * rows[br][i, sl]
            return carry

        lax.fori_loop(0, CHUNK // 4, mul, 0)
        wait_dst(bh)
        pltpu.async_copy(hevs[bh], acc.at[dstv[bh]], ssem[bh], add=True)

    def iteration(k, br, bh, bh_next, wait_sc):
        issue_gather(k + 1, 1 - br)
        if wait_sc:
            wait_scatter(bh_next)
        issue_he(k + 1, bh_next)
        compute_scatter(k, br, bh)

    issue_gather(0, 0)
    issue_he(0, 0)
    pltpu.async_copy(hv_hbm.at[srcv_all.at[pl.ds(NCH * CHUNK, TAIL)]],
                     rows_t, tg)
    pltpu.async_copy(he_hbm.at[pl.ds(ebase + NCH * CHUNK, TAIL)], hev_t, th)

    zero = jnp.zeros((16,), jnp.float32)

    def zbody(i, carry):
        for j in range(8):
            hev2[i, pl.ds(j * 16, 16)] = zero
        return carry

    lax.fori_loop(0, CHUNK, zbody, 0)
    base_r = s * ROWS_A

    @pl.when(s < NS - 1)
    def _():
        for k in range(9):
            pltpu.sync_copy(hev2, acc.at[pl.ds(base_r + k * CHUNK, CHUNK)])
        pltpu.sync_copy(hev2.at[pl.ds(0, ROWS_A - 9 * CHUNK)],
                        acc.at[pl.ds(base_r + 9 * CHUNK, ROWS_A - 9 * CHUNK)])

    @pl.when(s == NS - 1)
    def _():
        for k in range(10):
            pltpu.sync_copy(hev2, acc.at[pl.ds(base_r + k * CHUNK, CHUNK)])

    plsc.subcore_barrier()

    iteration(0, 0, 0, 1, False)
    iteration(1, 1, 1, 2, False)

    def steady(t, carry):
        k0 = STEADY0 + t * 6
        for d in range(6):
            k = k0 + d
            iteration(k, (STEADY0 + d) % NR, (STEADY0 + d) % NH,
                      (STEADY0 + d + 1) % NH, True)
        return carry

    lax.fori_loop(0, NSTEADY, steady, 0)
    k_hi = STEADY0 + NSTEADY * 6
    for k in range(k_hi, NCH - 1):
        iteration(k, k % NR, k % NH, (k + 1) % NH, True)
    compute_scatter(NCH - 1, (NCH - 1) % NR, (NCH - 1) % NH)
    for b in range(NH):
        wait_scatter(b)

    pltpu.make_async_copy(hv_hbm.at[srcv_all.at[pl.ds(NCH * CHUNK, TAIL)]],
                          rows_t, tg).wait()
    pltpu.make_async_copy(he_hbm.at[pl.ds(ebase, TAIL)], hev_t, th).wait()

    def mul_t(i, carry):
        for j in range(8):
            sl = pl.ds(j * 16, 16)
            hev_t[i, sl] = hev_t[i, sl] * rows_t[i, sl]
        return carry

    lax.fori_loop(0, TAIL, mul_t, 0)
    pltpu.async_copy(hev_t, acc.at[dstv_t], ts, add=True)
    pltpu.make_async_copy(hev_t, acc.at[dstv_t], ts).wait()

    plsc.subcore_barrier()

    @pl.when(s < NS - 1)
    def _():
        pltpu.sync_copy(acc.at[pl.ds(base_r, ROWS_A)],
                        out_hbm.at[c, pl.ds(base_r, ROWS_A)])

    @pl.when(s == NS - 1)
    def _():
        pltpu.sync_copy(acc.at[pl.ds(base_r, ROWS_LAST)],
                        out_hbm.at[c, pl.ds(base_r, ROWS_LAST)])


@functools.lru_cache(maxsize=1)
def _sc_gather_scatter_fn():
    return pl.kernel(
        _sc_body,
        out_type=jax.ShapeDtypeStruct((NC, N_NODES, EMB), jnp.float32),
        mesh=plsc.VectorSubcoreMesh(core_axis_name="c", subcore_axis_name="s",
                                    num_cores=NC, num_subcores=NS),
        scratch_types=[
            pltpu.VMEM((EDGES_PER_TILE,), jnp.int32),
            pltpu.VMEM((CHUNK,), jnp.int32),
            pltpu.VMEM((CHUNK,), jnp.int32),
            pltpu.VMEM((CHUNK,), jnp.int32),
            pltpu.VMEM((TAIL,), jnp.int32),
            pltpu.VMEM((CHUNK, EMB), jnp.float32),
            pltpu.VMEM((CHUNK, EMB), jnp.float32),
            pltpu.VMEM((CHUNK, EMB), jnp.float32),
            pltpu.VMEM((CHUNK, EMB), jnp.float32),
            pltpu.VMEM((CHUNK, EMB), jnp.float32),
            pltpu.VMEM((TAIL, EMB), jnp.float32),
            pltpu.VMEM((TAIL, EMB), jnp.float32),
            pltpu.VMEM_SHARED((N_NODES, EMB), jnp.float32),
        ] + [pltpu.SemaphoreType.DMA] * 14,
    )


def _sc_gather_scatter(hv, he, src, dst3, dtail):
    return _sc_gather_scatter_fn()(hv, he, src, dst3, dtail)


def kernel(nfeats, edge_index, efeats, embed, pn_W, pn_b, pe_W1, pe_b1, pe_W2,
           pe_b2, co_W, co_b, po_W, po_b, dec_W, dec_b, prelu_a):
    src = edge_index[0].astype(jnp.int32)
    dst = edge_index[1].astype(jnp.int32)
    nfeats = nfeats.astype(jnp.int32)
    dst_r = dst.reshape(NW, EDGES_PER_TILE)
    dst3 = dst_r[:, : NCH * CHUNK].reshape(NW, NCH, CHUNK)
    dtail = dst_r[:, NCH * CHUNK :]

    w1p = jnp.zeros((2, RBF_PAD, EMB), jnp.float32).at[:, :RBF_DIM, :].set(pe_W1)
    w1b = w1p.astype(jnp.bfloat16)
    w2b = pe_W2.astype(jnp.bfloat16)
    pa = jnp.broadcast_to(prelu_a[:, None], (4, EMB))

    he0, he1 = _edge_mlp(efeats, w1b, pe_b1, w2b, pe_b2)
    hv0 = _hv0(nfeats, embed, pn_W[0], pn_b[0][None, :])

    parts0 = _sc_gather_scatter(hv0, he0, src, dst3, dtail)
    hv1 = _mid(parts0[0], parts0[1], co_W[0], co_b[0][None, :], po_W[0],
               po_b[0][None, :], pn_W[1], pn_b[1][None, :])

    parts1 = _sc_gather_scatter(hv1, he1, src, dst3, dtail)
    out = _decode(parts1[0], parts1[1], co_W[1], co_b[1][None, :], po_W[1],
                  po_b[1][None, :], dec_W, dec_b, pa)
    return out


# speedup vs baseline: 1.2298x; 1.0772x over previous
import functools

import jax
import jax.numpy as jnp
from jax import lax
from jax.experimental import pallas as pl
from jax.experimental.pallas import tpu as pltpu
from jax.experimental.pallas import tpu_sc as plsc

N_NODES = 10000
N_EDGES = 160000
EMB = 128
RBF_DIM = 510
RBF_PAD = 512
CUTOFF = 51.0
INV_GAP = 10.0
LN2 = 0.6931471805599453

NC = 2
NS = 16
NW = NC * NS
EDGES_PER_TILE = N_EDGES // NW
CHUNK = 64
NCH = 78
TAIL = EDGES_PER_TILE - NCH * CHUNK
NR = 2
NH = 3
STEADY0 = 2
NSTEADY = 12
PEEL_HI = 4

ROWS_A = 624
ROWS_LAST = N_NODES - ROWS_A * (NS - 1)


def _ssp(x):
    return jnp.maximum(x, 0.0) + jnp.log(1.0 + jnp.exp(-jnp.abs(x))) - LN2


def _hv0_body(nt_ref, embed_ref, w_ref, b_ref, out_ref):
    nt = nt_ref[...]
    e0 = embed_ref[0, :][None, :]
    e1 = embed_ref[1, :][None, :]
    h0 = jnp.where(nt > 0, e1, e0)
    out_ref[...] = (
        jnp.dot(h0, w_ref[...], preferred_element_type=jnp.float32) + b_ref[...]
    )


GAPC = CUTOFF / (RBF_DIM - 1)


def _edge_body(ef_ref, w1_ref, b1_ref, w2_ref, b2_ref, he_ref):
    ef = ef_ref[...]
    sq = INV_GAP ** 0.5
    u = jnp.sqrt(jnp.sum(ef * ef, axis=1, keepdims=True)) * sq
    centers = lax.broadcasted_iota(jnp.int32, (1, RBF_PAD), 1).astype(
        jnp.float32
    ) * (GAPC * sq)
    rbf = jnp.exp(-((u - centers) ** 2)).astype(jnp.bfloat16)
    t = _ssp(
        jnp.dot(rbf, w1_ref[...], preferred_element_type=jnp.float32)
        + b1_ref[...]
    ).astype(jnp.bfloat16)
    t = _ssp(
        jnp.dot(t, w2_ref[...], preferred_element_type=jnp.float32) + b2_ref[...]
    )
    he_ref[...] = t


def _mid_body(p0_ref, p1_ref, cow_ref, cob_ref, pow_ref, pob_ref, pnw_ref,
              pnb_ref, out_ref):
    agg = p0_ref[...] + p1_ref[...]
    t = _ssp(jnp.dot(agg, cow_ref[...], preferred_element_type=jnp.float32)
             + cob_ref[...])
    h = jnp.dot(t, pow_ref[...], preferred_element_type=jnp.float32) + pob_ref[...]
    out_ref[...] = (
        jnp.dot(h, pnw_ref[...], preferred_element_type=jnp.float32) + pnb_ref[...]
    )


def _dec_body(p0_ref, p1_ref, cow_ref, cob_ref, pow_ref, pob_ref, decw_ref,
              decb_ref, pa_ref, out_ref):
    agg = p0_ref[...] + p1_ref[...]
    t = _ssp(jnp.dot(agg, cow_ref[...], preferred_element_type=jnp.float32)
             + cob_ref[...])
    x = jnp.dot(t, pow_ref[...], preferred_element_type=jnp.float32) + pob_ref[...]
    for j in range(4):
        x = jnp.dot(x, decw_ref[j], preferred_element_type=jnp.float32) + decb_ref[j]
        x = jnp.where(x >= 0, x, pa_ref[j] * x)
    out_ref[...] = (
        jnp.dot(x, decw_ref[4], preferred_element_type=jnp.float32) + decb_ref[4]
    )


def _full_spec(a):
    nd = a.ndim
    return pl.BlockSpec(a.shape, lambda i, _nd=nd: (0,) * _nd)


def _hv0(nfeats, embed, w, b):
    bn = 2000
    grid = (N_NODES // bn,)
    return pl.pallas_call(
        _hv0_body,
        grid=grid,
        in_specs=[
            pl.BlockSpec((bn, 1), lambda i: (i, 0)),
            _full_spec(embed),
            _full_spec(w),
            _full_spec(b),
        ],
        out_specs=pl.BlockSpec((bn, EMB), lambda i: (i, 0)),
        out_shape=jax.ShapeDtypeStruct((N_NODES, EMB), jnp.float32),
    )(nfeats, embed, w, b)


def _edge_mlp(efeats, w1p, b1, w2, b2):
    be = 1600
    grid = (N_EDGES // be,)
    return pl.pallas_call(
        _edge_body,
        grid=grid,
        in_specs=[
            pl.BlockSpec((be, 16), lambda i: (i, 0)),
            _full_spec(w1p),
            _full_spec(b1),
            _full_spec(w2),
            _full_spec(b2),
        ],
        out_specs=pl.BlockSpec((be, EMB), lambda i: (i, 0)),
        out_shape=jax.ShapeDtypeStruct((N_EDGES, EMB), jnp.float32),
    )(efeats, w1p, b1, w2, b2)


def _mid(p0, p1, cow, cob, pow_, pob, pnw, pnb):
    bn = 2000
    grid = (N_NODES // bn,)
    return pl.pallas_call(
        _mid_body,
        grid=grid,
        in_specs=[
            pl.BlockSpec((bn, EMB), lambda i: (i, 0)),
            pl.BlockSpec((bn, EMB), lambda i: (i, 0)),
            _full_spec(cow), _full_spec(cob),
            _full_spec(pow_), _full_spec(pob),
            _full_spec(pnw), _full_spec(pnb),
        ],
        out_specs=pl.BlockSpec((bn, EMB), lambda i: (i, 0)),
        out_shape=jax.ShapeDtypeStruct((N_NODES, EMB), jnp.float32),
    )(p0, p1, cow, cob, pow_, pob, pnw, pnb)


def _decode(p0, p1, cow, cob, pow_, pob, decw, decb, pa):
    bn = 2000
    grid = (N_NODES // bn,)
    return pl.pallas_call(
        _dec_body,
        grid=grid,
        in_specs=[
            pl.BlockSpec((bn, EMB), lambda i: (i, 0)),
            pl.BlockSpec((bn, EMB), lambda i: (i, 0)),
            _full_spec(cow), _full_spec(cob),
            _full_spec(pow_), _full_spec(pob),
            _full_spec(decw), _full_spec(decb), _full_spec(pa),
        ],
        out_specs=pl.BlockSpec((bn, EMB), lambda i: (i, 0)),
        out_shape=jax.ShapeDtypeStruct((N_NODES, EMB), jnp.float32),
    )(p0, p1, cow, cob, pow_, pob, decw, decb, pa)


def _sc_body(hv_hbm, he_hbm, src_hbm, dst3_hbm, dtail_hbm, out_hbm,
             srcv_all, dstv0, dstv1, dstv2, dstv_t,
             rows0, rows1, hev0, hev1, hev2, rows_t, hev_t, acc,
             g0, g1, h0, h1, h2, s0, s1, s2, d0, d1, d2, tg, th, ts):
    c = lax.axis_index("c")
    s = lax.axis_index("s")
    gtile = c * NS + s
    ebase = gtile * EDGES_PER_TILE

    rows = (rows0, rows1)
    hevs = (hev0, hev1, hev2)
    dstv = (dstv0, dstv1, dstv2)
    gsem = (g0, g1)
    hsem = (h0, h1, h2)
    ssem = (s0, s1, s2)
    dsem = (d0, d1, d2)

    pltpu.sync_copy(src_hbm.at[pl.ds(ebase, EDGES_PER_TILE)], srcv_all)
    pltpu.sync_copy(dtail_hbm.at[gtile], dstv_t)

    def issue_gather(k, br):
        pltpu.async_copy(hv_hbm.at[srcv_all.at[pl.ds(k * CHUNK, CHUNK)]],
                         rows[br], gsem[br])

    def issue_he(k, bh):
        pltpu.async_copy(he_hbm.at[pl.ds(ebase + k * CHUNK, CHUNK)],
                         hevs[bh], hsem[bh])
        pltpu.async_copy(dst3_hbm.at[gtile, k], dstv[bh], dsem[bh])

    def wait_gather(br):
        pltpu.make_async_copy(hv_hbm.at[srcv_all.at[pl.ds(0, CHUNK)]],
                              rows[br], gsem[br]).wait()

    def wait_he(bh):
        pltpu.make_async_copy(he_hbm.at[pl.ds(ebase, CHUNK)],
                              hevs[bh], hsem[bh]).wait()

    def wait_dst(bh):
        pltpu.make_async_copy(dst3_hbm.at[gtile, 0], dstv[bh],
                              dsem[bh]).wait()

    def wait_scatter(bh):
        pltpu.make_async_copy(hevs[bh], acc.at[dstv[bh]], ssem[bh]).wait()

    def compute_scatter(k, br, bh):
        wait_gather(br)
        wait_he(bh)

        def mul(i4, carry):
            for r in range(4):
                i = i4 * 4 + r
                for j in range(8):
                    sl = pl.ds(j * 16, 16)
                    hevs[bh][i, sl] = hevs[bh][i, sl] * rows[br][i, sl]
            return carry

        lax.fori_loop(0, CHUNK // 4, mul, 0)
        wait_dst(bh)
        pltpu.async_copy(hevs[bh], acc.at[dstv[bh]], ssem[bh], add=True)

    def iteration(k, br, bh, bh_next, wait_sc):
        issue_gather(k + 1, 1 - br)
        if wait_sc:
            wait_scatter(bh_next)
        issue_he(k + 1, bh_next)
        compute_scatter(k, br, bh)

    issue_gather(0, 0)
    issue_he(0, 0)
    pltpu.async_copy(hv_hbm.at[srcv_all.at[pl.ds(NCH * CHUNK, TAIL)]],
                     rows_t, tg)
    pltpu.async_copy(he_hbm.at[pl.ds(ebase + NCH * CHUNK, TAIL)], hev_t, th)

    zero = jnp.zeros((16,), jnp.float32)

    def zbody(i, carry):
        for j in range(8):
            hev2[i, pl.ds(j * 16, 16)] = zero
        return carry

    lax.fori_loop(0, CHUNK, zbody, 0)
    base_r = s * ROWS_A

    @pl.when(s < NS - 1)
    def _():
        for k in range(9):
            pltpu.sync_copy(hev2, acc.at[pl.ds(base_r + k * CHUNK, CHUNK)])
        pltpu.sync_copy(hev2.at[pl.ds(0, ROWS_A - 9 * CHUNK)],
                        acc.at[pl.ds(base_r + 9 * CHUNK, ROWS_A - 9 * CHUNK)])

    @pl.when(s == NS - 1)
    def _():
        for k in range(10):
            pltpu.sync_copy(hev2, acc.at[pl.ds(base_r + k * CHUNK, CHUNK)])

    plsc.subcore_barrier()

    iteration(0, 0, 0, 1, False)
    iteration(1, 1, 1, 2, False)

    def steady(t, carry):
        k0 = STEADY0 + t * 6
        for d in range(6):
            k = k0 + d
            iteration(k, (STEADY0 + d) % NR, (STEADY0 + d) % NH,
                      (STEADY0 + d + 1) % NH, True)
        return carry

    lax.fori_loop(0, NSTEADY, steady, 0)
    k_hi = STEADY0 + NSTEADY * 6
    for k in range(k_hi, NCH - 1):
        iteration(k, k % NR, k % NH, (k + 1) % NH, True)
    compute_scatter(NCH - 1, (NCH - 1) % NR, (NCH - 1) % NH)
    for b in range(NH):
        wait_scatter(b)

    pltpu.make_async_copy(hv_hbm.at[srcv_all.at[pl.ds(NCH * CHUNK, TAIL)]],
                          rows_t, tg).wait()
    pltpu.make_async_copy(he_hbm.at[pl.ds(ebase, TAIL)], hev_t, th).wait()

    def mul_t(i, carry):
        for j in range(8):
            sl = pl.ds(j * 16, 16)
            hev_t[i, sl] = hev_t[i, sl] * rows_t[i, sl]
        return carry

    lax.fori_loop(0, TAIL, mul_t, 0)
    pltpu.async_copy(hev_t, acc.at[dstv_t], ts, add=True)
    pltpu.make_async_copy(hev_t, acc.at[dstv_t], ts).wait()

    plsc.subcore_barrier()

    @pl.when(s < NS - 1)
    def _():
        pltpu.sync_copy(acc.at[pl.ds(base_r, ROWS_A)],
                        out_hbm.at[c, pl.ds(base_r, ROWS_A)])

    @pl.when(s == NS - 1)
    def _():
        pltpu.sync_copy(acc.at[pl.ds(base_r, ROWS_LAST)],
                        out_hbm.at[c, pl.ds(base_r, ROWS_LAST)])


@functools.lru_cache(maxsize=1)
def _sc_gather_scatter_fn():
    return pl.kernel(
        _sc_body,
        out_type=jax.ShapeDtypeStruct((NC, N_NODES, EMB), jnp.float32),
        mesh=plsc.VectorSubcoreMesh(core_axis_name="c", subcore_axis_name="s",
                                    num_cores=NC, num_subcores=NS),
        scratch_types=[
            pltpu.VMEM((EDGES_PER_TILE,), jnp.int32),
            pltpu.VMEM((CHUNK,), jnp.int32),
            pltpu.VMEM((CHUNK,), jnp.int32),
            pltpu.VMEM((CHUNK,), jnp.int32),
            pltpu.VMEM((TAIL,), jnp.int32),
            pltpu.VMEM((CHUNK, EMB), jnp.float32),
            pltpu.VMEM((CHUNK, EMB), jnp.float32),
            pltpu.VMEM((CHUNK, EMB), jnp.float32),
            pltpu.VMEM((CHUNK, EMB), jnp.float32),
            pltpu.VMEM((CHUNK, EMB), jnp.float32),
            pltpu.VMEM((TAIL, EMB), jnp.float32),
            pltpu.VMEM((TAIL, EMB), jnp.float32),
            pltpu.VMEM_SHARED((N_NODES, EMB), jnp.float32),
        ] + [pltpu.SemaphoreType.DMA] * 14,
    )


def _sc_gather_scatter(hv, he, src, dst3, dtail):
    return _sc_gather_scatter_fn()(hv, he, src, dst3, dtail)


def kernel(nfeats, edge_index, efeats, embed, pn_W, pn_b, pe_W1, pe_b1, pe_W2,
           pe_b2, co_W, co_b, po_W, po_b, dec_W, dec_b, prelu_a):
    src = edge_index[0].astype(jnp.int32)
    dst = edge_index[1].astype(jnp.int32)
    nfeats = nfeats.astype(jnp.int32)
    dst_r = dst.reshape(NW, EDGES_PER_TILE)
    dst3 = dst_r[:, : NCH * CHUNK].reshape(NW, NCH, CHUNK)
    dtail = dst_r[:, NCH * CHUNK :]

    w1p = jnp.zeros((2, RBF_PAD, EMB), jnp.float32).at[:, :RBF_DIM, :].set(pe_W1)
    w1b = w1p.astype(jnp.bfloat16)
    w2b = pe_W2.astype(jnp.bfloat16)
    pa = jnp.broadcast_to(prelu_a[:, None], (4, EMB))

    he0 = _edge_mlp(efeats, w1b[0], pe_b1[0][None, :], w2b[0], pe_b2[0][None, :])
    hv0 = _hv0(nfeats, embed, pn_W[0], pn_b[0][None, :])

    parts0 = _sc_gather_scatter(hv0, he0, src, dst3, dtail)
    he1 = _edge_mlp(efeats, w1b[1], pe_b1[1][None, :], w2b[1], pe_b2[1][None, :])
    hv1 = _mid(parts0[0], parts0[1], co_W[0], co_b[0][None, :], po_W[0],
               po_b[0][None, :], pn_W[1], pn_b[1][None, :])

    parts1 = _sc_gather_scatter(hv1, he1, src, dst3, dtail)
    out = _decode(parts1[0], parts1[1], co_W[1], co_b[1][None, :], po_W[1],
                  po_b[1][None, :], dec_W, dec_b, pa)
    return out


# speedup vs baseline: 1.2767x; 1.0382x over previous
import functools

import jax
import jax.numpy as jnp
from jax import lax
from jax.experimental import pallas as pl
from jax.experimental.pallas import tpu as pltpu
from jax.experimental.pallas import tpu_sc as plsc

N_NODES = 10000
N_EDGES = 160000
EMB = 128
RBF_DIM = 510
RBF_PAD = 512
CUTOFF = 51.0
INV_GAP = 10.0
LN2 = 0.6931471805599453

NC = 2
NS = 16
NW = NC * NS
EDGES_PER_TILE = N_EDGES // NW
CHUNK = 64
NCH = 78
TAIL = EDGES_PER_TILE - NCH * CHUNK
NR = 2
NH = 3
STEADY0 = 2
NSTEADY = 12
PEEL_HI = 4

ROWS_A = 624
ROWS_LAST = N_NODES - ROWS_A * (NS - 1)


def _ssp(x):
    return jnp.maximum(x, 0.0) + jnp.log(1.0 + jnp.exp(-jnp.abs(x))) - LN2


def _hv0_body(nt_ref, embed_ref, w_ref, b_ref, out_ref):
    nt = nt_ref[...]
    e0 = embed_ref[0, :][None, :]
    e1 = embed_ref[1, :][None, :]
    h0 = jnp.where(nt > 0, e1, e0)
    out_ref[...] = (
        jnp.dot(h0, w_ref[...], preferred_element_type=jnp.float32) + b_ref[...]
    )


GAPC = CUTOFF / (RBF_DIM - 1)


WIN = 256
REACH = 2.0


def _edge_body(ef_ref, w1_ref, b1_ref, w2_ref, b2_ref, he_ref):
    ef = ef_ref[...]
    sq = INV_GAP ** 0.5
    u = jnp.sqrt(jnp.sum(ef * ef, axis=1, keepdims=True)) * sq
    d = u * (1.0 / sq)
    dmin = jnp.min(d)
    dmax = jnp.max(d)

    def _tail(rbf, w1, b1):
        t = _ssp(jnp.dot(rbf, w1, preferred_element_type=jnp.float32)
                 + b1).astype(jnp.bfloat16)
        t = _ssp(jnp.dot(t, w2_ref[...], preferred_element_type=jnp.float32)
                 + b2_ref[...])
        he_ref[...] = t

    narrow = (dmax - dmin) <= (WIN - 1) * GAPC - 2.0 * REACH

    @pl.when(narrow)
    def _():
        s_f = jnp.floor((dmin - REACH) / (8.0 * GAPC)) * 8.0
        start = jnp.clip(s_f.astype(jnp.int32), 0, RBF_PAD - WIN)
        start = pl.multiple_of(start, 8)
        cix = lax.broadcasted_iota(jnp.int32, (1, WIN), 1).astype(jnp.float32)
        centers = (cix + start.astype(jnp.float32)) * (GAPC * sq)
        rbf = jnp.exp(-((u - centers) ** 2)).astype(jnp.bfloat16)
        _tail(rbf, w1_ref[pl.ds(start, WIN), :], b1_ref[...])

    @pl.when(jnp.logical_not(narrow))
    def _():
        cix = lax.broadcasted_iota(jnp.int32, (1, RBF_PAD), 1).astype(
            jnp.float32)
        centers = cix * (GAPC * sq)
        rbf = jnp.exp(-((u - centers) ** 2)).astype(jnp.bfloat16)
        _tail(rbf, w1_ref[...], b1_ref[...])


def _mid_body(p0_ref, p1_ref, cow_ref, cob_ref, pow_ref, pob_ref, pnw_ref,
              pnb_ref, out_ref):
    agg = p0_ref[...] + p1_ref[...]
    t = _ssp(jnp.dot(agg, cow_ref[...], preferred_element_type=jnp.float32)
             + cob_ref[...])
    h = jnp.dot(t, pow_ref[...], preferred_element_type=jnp.float32) + pob_ref[...]
    out_ref[...] = (
        jnp.dot(h, pnw_ref[...], preferred_element_type=jnp.float32) + pnb_ref[...]
    )


def _dec_body(p0_ref, p1_ref, cow_ref, cob_ref, pow_ref, pob_ref, decw_ref,
              decb_ref, pa_ref, out_ref):
    agg = p0_ref[...] + p1_ref[...]
    t = _ssp(jnp.dot(agg, cow_ref[...], preferred_element_type=jnp.float32)
             + cob_ref[...])
    x = jnp.dot(t, pow_ref[...], preferred_element_type=jnp.float32) + pob_ref[...]
    for j in range(4):
        x = jnp.dot(x, decw_ref[j], preferred_element_type=jnp.float32) + decb_ref[j]
        x = jnp.where(x >= 0, x, pa_ref[j] * x)
    out_ref[...] = (
        jnp.dot(x, decw_ref[4], preferred_element_type=jnp.float32) + decb_ref[4]
    )


def _full_spec(a):
    nd = a.ndim
    return pl.BlockSpec(a.shape, lambda i, _nd=nd: (0,) * _nd)


def _hv0(nfeats, embed, w, b):
    bn = 2000
    grid = (N_NODES // bn,)
    return pl.pallas_call(
        _hv0_body,
        grid=grid,
        in_specs=[
            pl.BlockSpec((bn, 1), lambda i: (i, 0)),
            _full_spec(embed),
            _full_spec(w),
            _full_spec(b),
        ],
        out_specs=pl.BlockSpec((bn, EMB), lambda i: (i, 0)),
        out_shape=jax.ShapeDtypeStruct((N_NODES, EMB), jnp.float32),
    )(nfeats, embed, w, b)


def _edge_mlp(efeats, w1p, b1, w2, b2):
    be = 1600
    grid = (N_EDGES // be,)
    return pl.pallas_call(
        _edge_body,
        grid=grid,
        in_specs=[
            pl.BlockSpec((be, 16), lambda i: (i, 0)),
            _full_spec(w1p),
            _full_spec(b1),
            _full_spec(w2),
            _full_spec(b2),
        ],
        out_specs=pl.BlockSpec((be, EMB), lambda i: (i, 0)),
        out_shape=jax.ShapeDtypeStruct((N_EDGES, EMB), jnp.float32),
    )(efeats, w1p, b1, w2, b2)


def _mid(p0, p1, cow, cob, pow_, pob, pnw, pnb):
    bn = 2000
    grid = (N_NODES // bn,)
    return pl.pallas_call(
        _mid_body,
        grid=grid,
        in_specs=[
            pl.BlockSpec((bn, EMB), lambda i: (i, 0)),
            pl.BlockSpec((bn, EMB), lambda i: (i, 0)),
            _full_spec(cow), _full_spec(cob),
            _full_spec(pow_), _full_spec(pob),
            _full_spec(pnw), _full_spec(pnb),
        ],
        out_specs=pl.BlockSpec((bn, EMB), lambda i: (i, 0)),
        out_shape=jax.ShapeDtypeStruct((N_NODES, EMB), jnp.float32),
    )(p0, p1, cow, cob, pow_, pob, pnw, pnb)


def _decode(p0, p1, cow, cob, pow_, pob, decw, decb, pa):
    bn = 2000
    grid = (N_NODES // bn,)
    return pl.pallas_call(
        _dec_body,
        grid=grid,
        in_specs=[
            pl.BlockSpec((bn, EMB), lambda i: (i, 0)),
            pl.BlockSpec((bn, EMB), lambda i: (i, 0)),
            _full_spec(cow), _full_spec(cob),
            _full_spec(pow_), _full_spec(pob),
            _full_spec(decw), _full_spec(decb), _full_spec(pa),
        ],
        out_specs=pl.BlockSpec((bn, EMB), lambda i: (i, 0)),
        out_shape=jax.ShapeDtypeStruct((N_NODES, EMB), jnp.float32),
    )(p0, p1, cow, cob, pow_, pob, decw, decb, pa)


def _sc_body(hv_hbm, he_hbm, src_hbm, dst3_hbm, dtail_hbm, out_hbm,
             srcv_all, dstv0, dstv1, dstv2, dstv_t,
             rows0, rows1, hev0, hev1, hev2, rows_t, hev_t, acc,
             g0, g1, h0, h1, h2, s0, s1, s2, d0, d1, d2, tg, th, ts):
    c = lax.axis_index("c")
    s = lax.axis_index("s")
    gtile = c * NS + s
    ebase = gtile * EDGES_PER_TILE

    rows = (rows0, rows1)
    hevs = (hev0, hev1, hev2)
    dstv = (dstv0, dstv1, dstv2)
    gsem = (g0, g1)
    hsem = (h0, h1, h2)
    ssem = (s0, s1, s2)
    dsem = (d0, d1, d2)

    pltpu.sync_copy(src_hbm.at[pl.ds(ebase, EDGES_PER_TILE)], srcv_all)
    pltpu.sync_copy(dtail_hbm.at[gtile], dstv_t)

    def issue_gather(k, br):
        pltpu.async_copy(hv_hbm.at[srcv_all.at[pl.ds(k * CHUNK, CHUNK)]],
                         rows[br], gsem[br])

    def issue_he(k, bh):
        pltpu.async_copy(he_hbm.at[pl.ds(ebase + k * CHUNK, CHUNK)],
                         hevs[bh], hsem[bh])
        pltpu.async_copy(dst3_hbm.at[gtile, k], dstv[bh], dsem[bh])

    def wait_gather(br):
        pltpu.make_async_copy(hv_hbm.at[srcv_all.at[pl.ds(0, CHUNK)]],
                              rows[br], gsem[br]).wait()

    def wait_he(bh):
        pltpu.make_async_copy(he_hbm.at[pl.ds(ebase, CHUNK)],
                              hevs[bh], hsem[bh]).wait()

    def wait_dst(bh):
        pltpu.make_async_copy(dst3_hbm.at[gtile, 0], dstv[bh],
                              dsem[bh]).wait()

    def wait_scatter(bh):
        pltpu.make_async_copy(hevs[bh], acc.at[dstv[bh]], ssem[bh]).wait()

    def compute_scatter(k, br, bh):
        wait_gather(br)
        wait_he(bh)

        def mul(i4, carry):
            for r in range(4):
                i = i4 * 4 + r
                for j in range(8):
                    sl = pl.ds(j * 16, 16)
                    hevs[bh][i, sl] = hevs[bh][i, sl] * rows[br][i, sl]
            return carry

        lax.fori_loop(0, CHUNK // 4, mul, 0)
        wait_dst(bh)
        pltpu.async_copy(hevs[bh], acc.at[dstv[bh]], ssem[bh], add=True)

    def iteration(k, br, bh, bh_next, wait_sc):
        issue_gather(k + 1, 1 - br)
        if wait_sc:
            wait_scatter(bh_next)
        issue_he(k + 1, bh_next)
        compute_scatter(k, br, bh)

    issue_gather(0, 0)
    issue_he(0, 0)
    pltpu.async_copy(hv_hbm.at[srcv_all.at[pl.ds(NCH * CHUNK, TAIL)]],
                     rows_t, tg)
    pltpu.async_copy(he_hbm.at[pl.ds(ebase + NCH * CHUNK, TAIL)], hev_t, th)

    zero = jnp.zeros((16,), jnp.float32)

    def zbody(i, carry):
        for j in range(8):
            hev2[i, pl.ds(j * 16, 16)] = zero
        return carry

    lax.fori_loop(0, CHUNK, zbody, 0)
    base_r = s * ROWS_A

    @pl.when(s < NS - 1)
    def _():
        for k in range(9):
            pltpu.sync_copy(hev2, acc.at[pl.ds(base_r + k * CHUNK, CHUNK)])
        pltpu.sync_copy(hev2.at[pl.ds(0, ROWS_A - 9 * CHUNK)],
                        acc.at[pl.ds(base_r + 9 * CHUNK, ROWS_A - 9 * CHUNK)])

    @pl.when(s == NS - 1)
    def _():
        for k in range(10):
            pltpu.sync_copy(hev2, acc.at[pl.ds(base_r + k * CHUNK, CHUNK)])

    plsc.subcore_barrier()

    iteration(0, 0, 0, 1, False)
    iteration(1, 1, 1, 2, False)

    def steady(t, carry):
        k0 = STEADY0 + t * 6
        for d in range(6):
            k = k0 + d
            iteration(k, (STEADY0 + d) % NR, (STEADY0 + d) % NH,
                      (STEADY0 + d + 1) % NH, True)
        return carry

    lax.fori_loop(0, NSTEADY, steady, 0)
    k_hi = STEADY0 + NSTEADY * 6
    for k in range(k_hi, NCH - 1):
        iteration(k, k % NR, k % NH, (k + 1) % NH, True)
    compute_scatter(NCH - 1, (NCH - 1) % NR, (NCH - 1) % NH)
    for b in range(NH):
        wait_scatter(b)

    pltpu.make_async_copy(hv_hbm.at[srcv_all.at[pl.ds(NCH * CHUNK, TAIL)]],
                          rows_t, tg).wait()
    pltpu.make_async_copy(he_hbm.at[pl.ds(ebase, TAIL)], hev_t, th).wait()

    def mul_t(i, carry):
        for j in range(8):
            sl = pl.ds(j * 16, 16)
            hev_t[i, sl] = hev_t[i, sl] * rows_t[i, sl]
        return carry

    lax.fori_loop(0, TAIL, mul_t, 0)
    pltpu.async_copy(hev_t, acc.at[dstv_t], ts, add=True)
    pltpu.make_async_copy(hev_t, acc.at[dstv_t], ts).wait()

    plsc.subcore_barrier()

    @pl.when(s < NS - 1)
    def _():
        pltpu.sync_copy(acc.at[pl.ds(base_r, ROWS_A)],
                        out_hbm.at[c, pl.ds(base_r, ROWS_A)])

    @pl.when(s == NS - 1)
    def _():
        pltpu.sync_copy(acc.at[pl.ds(base_r, ROWS_LAST)],
                        out_hbm.at[c, pl.ds(base_r, ROWS_LAST)])


@functools.lru_cache(maxsize=1)
def _sc_gather_scatter_fn():
    return pl.kernel(
        _sc_body,
        out_type=jax.ShapeDtypeStruct((NC, N_NODES, EMB), jnp.float32),
        mesh=plsc.VectorSubcoreMesh(core_axis_name="c", subcore_axis_name="s",
                                    num_cores=NC, num_subcores=NS),
        scratch_types=[
            pltpu.VMEM((EDGES_PER_TILE,), jnp.int32),
            pltpu.VMEM((CHUNK,), jnp.int32),
            pltpu.VMEM((CHUNK,), jnp.int32),
            pltpu.VMEM((CHUNK,), jnp.int32),
            pltpu.VMEM((TAIL,), jnp.int32),
            pltpu.VMEM((CHUNK, EMB), jnp.float32),
            pltpu.VMEM((CHUNK, EMB), jnp.float32),
            pltpu.VMEM((CHUNK, EMB), jnp.float32),
            pltpu.VMEM((CHUNK, EMB), jnp.float32),
            pltpu.VMEM((CHUNK, EMB), jnp.float32),
            pltpu.VMEM((TAIL, EMB), jnp.float32),
            pltpu.VMEM((TAIL, EMB), jnp.float32),
            pltpu.VMEM_SHARED((N_NODES, EMB), jnp.float32),
        ] + [pltpu.SemaphoreType.DMA] * 14,
    )


def _sc_gather_scatter(hv, he, src, dst3, dtail):
    return _sc_gather_scatter_fn()(hv, he, src, dst3, dtail)


def kernel(nfeats, edge_index, efeats, embed, pn_W, pn_b, pe_W1, pe_b1, pe_W2,
           pe_b2, co_W, co_b, po_W, po_b, dec_W, dec_b, prelu_a):
    src = edge_index[0].astype(jnp.int32)
    dst = edge_index[1].astype(jnp.int32)
    nfeats = nfeats.astype(jnp.int32)
    dst_r = dst.reshape(NW, EDGES_PER_TILE)
    dst3 = dst_r[:, : NCH * CHUNK].reshape(NW, NCH, CHUNK)
    dtail = dst_r[:, NCH * CHUNK :]

    w1p = jnp.zeros((2, RBF_PAD, EMB), jnp.float32).at[:, :RBF_DIM, :].set(pe_W1)
    w1b = w1p.astype(jnp.bfloat16)
    w2b = pe_W2.astype(jnp.bfloat16)
    pa = jnp.broadcast_to(prelu_a[:, None], (4, EMB))

    he0 = _edge_mlp(efeats, w1b[0], pe_b1[0][None, :], w2b[0], pe_b2[0][None, :])
    hv0 = _hv0(nfeats, embed, pn_W[0], pn_b[0][None, :])

    parts0 = _sc_gather_scatter(hv0, he0, src, dst3, dtail)
    he1 = _edge_mlp(efeats, w1b[1], pe_b1[1][None, :], w2b[1], pe_b2[1][None, :])
    hv1 = _mid(parts0[0], parts0[1], co_W[0], co_b[0][None, :], po_W[0],
               po_b[0][None, :], pn_W[1], pn_b[1][None, :])

    parts1 = _sc_gather_scatter(hv1, he1, src, dst3, dtail)
    out = _decode(parts1[0], parts1[1], co_W[1], co_b[1][None, :], po_W[1],
                  po_b[1][None, :], dec_W, dec_b, pa)
    return out


# speedup vs baseline: 1.3150x; 1.0300x over previous
import functools

import jax
import jax.numpy as jnp
from jax import lax
from jax.experimental import pallas as pl
from jax.experimental.pallas import tpu as pltpu
from jax.experimental.pallas import tpu_sc as plsc

N_NODES = 10000
N_EDGES = 160000
EMB = 128
RBF_DIM = 510
RBF_PAD = 512
CUTOFF = 51.0
INV_GAP = 10.0
LN2 = 0.6931471805599453

NC = 2
NS = 16
NW = NC * NS
EDGES_PER_TILE = N_EDGES // NW
CHUNK = 64
NCH = 78
TAIL = EDGES_PER_TILE - NCH * CHUNK
NR = 2
NH = 3
STEADY0 = 2
NSTEADY = 12
PEEL_HI = 4

ROWS_A = 624
ROWS_LAST = N_NODES - ROWS_A * (NS - 1)


def _ssp(x):
    return jnp.maximum(x, 0.0) + jnp.log(1.0 + jnp.exp(-jnp.abs(x))) - LN2


def _hv0_body(nt_ref, embed_ref, w_ref, b_ref, out_ref):
    nt = nt_ref[...]
    e0 = embed_ref[0, :][None, :]
    e1 = embed_ref[1, :][None, :]
    h0 = jnp.where(nt > 0, e1, e0)
    out_ref[...] = (
        jnp.dot(h0, w_ref[...], preferred_element_type=jnp.float32) + b_ref[...]
    )


GAPC = CUTOFF / (RBF_DIM - 1)


WIN = 128
REACH = 2.0


def _edge_body(ef_ref, w1_ref, b1_ref, w2_ref, b2_ref, he_ref):
    ef = ef_ref[...]
    sq = INV_GAP ** 0.5
    u = jnp.sqrt(jnp.sum(ef * ef, axis=1, keepdims=True)) * sq
    d = u * (1.0 / sq)
    dmin = jnp.min(d)
    dmax = jnp.max(d)

    def _tail(rbf, w1, b1):
        t = _ssp(jnp.dot(rbf, w1, preferred_element_type=jnp.float32)
                 + b1).astype(jnp.bfloat16)
        t = _ssp(jnp.dot(t, w2_ref[...], preferred_element_type=jnp.float32)
                 + b2_ref[...])
        he_ref[...] = t

    narrow = (dmax - dmin) <= (WIN - 9) * GAPC - 2.0 * REACH

    @pl.when(narrow)
    def _():
        s_f = jnp.floor((dmin - REACH) / (8.0 * GAPC)) * 8.0
        start = jnp.clip(s_f.astype(jnp.int32), 0, RBF_PAD - WIN)
        start = pl.multiple_of(start, 8)
        cix = lax.broadcasted_iota(jnp.int32, (1, WIN), 1).astype(jnp.float32)
        centers = (cix + start.astype(jnp.float32)) * (GAPC * sq)
        rbf = jnp.exp(-((u - centers) ** 2)).astype(jnp.bfloat16)
        _tail(rbf, w1_ref[pl.ds(start, WIN), :], b1_ref[...])

    @pl.when(jnp.logical_not(narrow))
    def _():
        cix = lax.broadcasted_iota(jnp.int32, (1, RBF_PAD), 1).astype(
            jnp.float32)
        centers = cix * (GAPC * sq)
        rbf = jnp.exp(-((u - centers) ** 2)).astype(jnp.bfloat16)
        _tail(rbf, w1_ref[...], b1_ref[...])


def _mid_body(p0_ref, p1_ref, cow_ref, cob_ref, pow_ref, pob_ref, pnw_ref,
              pnb_ref, out_ref):
    agg = p0_ref[...] + p1_ref[...]
    t = _ssp(jnp.dot(agg, cow_ref[...], preferred_element_type=jnp.float32)
             + cob_ref[...])
    h = jnp.dot(t, pow_ref[...], preferred_element_type=jnp.float32) + pob_ref[...]
    out_ref[...] = (
        jnp.dot(h, pnw_ref[...], preferred_element_type=jnp.float32) + pnb_ref[...]
    )


def _dec_body(p0_ref, p1_ref, cow_ref, cob_ref, pow_ref, pob_ref, decw_ref,
              decb_ref, pa_ref, out_ref):
    agg = p0_ref[...] + p1_ref[...]
    t = _ssp(jnp.dot(agg, cow_ref[...], preferred_element_type=jnp.float32)
             + cob_ref[...])
    x = jnp.dot(t, pow_ref[...], preferred_element_type=jnp.float32) + pob_ref[...]
    for j in range(4):
        x = jnp.dot(x, decw_ref[j], preferred_element_type=jnp.float32) + decb_ref[j]
        x = jnp.where(x >= 0, x, pa_ref[j] * x)
    out_ref[...] = (
        jnp.dot(x, decw_ref[4], preferred_element_type=jnp.float32) + decb_ref[4]
    )


def _full_spec(a):
    nd = a.ndim
    return pl.BlockSpec(a.shape, lambda i, _nd=nd: (0,) * _nd)


def _hv0(nfeats, embed, w, b):
    bn = 2000
    grid = (N_NODES // bn,)
    return pl.pallas_call(
        _hv0_body,
        grid=grid,
        in_specs=[
            pl.BlockSpec((bn, 1), lambda i: (i, 0)),
            _full_spec(embed),
            _full_spec(w),
            _full_spec(b),
        ],
        out_specs=pl.BlockSpec((bn, EMB), lambda i: (i, 0)),
        out_shape=jax.ShapeDtypeStruct((N_NODES, EMB), jnp.float32),
    )(nfeats, embed, w, b)


def _edge_mlp(efeats, w1p, b1, w2, b2):
    be = 1600
    grid = (N_EDGES // be,)
    return pl.pallas_call(
        _edge_body,
        grid=grid,
        in_specs=[
            pl.BlockSpec((be, 16), lambda i: (i, 0)),
            _full_spec(w1p),
            _full_spec(b1),
            _full_spec(w2),
            _full_spec(b2),
        ],
        out_specs=pl.BlockSpec((be, EMB), lambda i: (i, 0)),
        out_shape=jax.ShapeDtypeStruct((N_EDGES, EMB), jnp.float32),
    )(efeats, w1p, b1, w2, b2)


def _mid(p0, p1, cow, cob, pow_, pob, pnw, pnb):
    bn = 2000
    grid = (N_NODES // bn,)
    return pl.pallas_call(
        _mid_body,
        grid=grid,
        in_specs=[
            pl.BlockSpec((bn, EMB), lambda i: (i, 0)),
            pl.BlockSpec((bn, EMB), lambda i: (i, 0)),
            _full_spec(cow), _full_spec(cob),
            _full_spec(pow_), _full_spec(pob),
            _full_spec(pnw), _full_spec(pnb),
        ],
        out_specs=pl.BlockSpec((bn, EMB), lambda i: (i, 0)),
        out_shape=jax.ShapeDtypeStruct((N_NODES, EMB), jnp.float32),
    )(p0, p1, cow, cob, pow_, pob, pnw, pnb)


def _decode(p0, p1, cow, cob, pow_, pob, decw, decb, pa):
    bn = 2000
    grid = (N_NODES // bn,)
    return pl.pallas_call(
        _dec_body,
        grid=grid,
        in_specs=[
            pl.BlockSpec((bn, EMB), lambda i: (i, 0)),
            pl.BlockSpec((bn, EMB), lambda i: (i, 0)),
            _full_spec(cow), _full_spec(cob),
            _full_spec(pow_), _full_spec(pob),
            _full_spec(decw), _full_spec(decb), _full_spec(pa),
        ],
        out_specs=pl.BlockSpec((bn, EMB), lambda i: (i, 0)),
        out_shape=jax.ShapeDtypeStruct((N_NODES, EMB), jnp.float32),
    )(p0, p1, cow, cob, pow_, pob, decw, decb, pa)


def _sc_body(hv_hbm, he_hbm, src_hbm, dst3_hbm, dtail_hbm, out_hbm,
             srcv_all, dstv0, dstv1, dstv2, dstv_t,
             rows0, rows1, hev0, hev1, hev2, rows_t, hev_t, acc,
             g0, g1, h0, h1, h2, s0, s1, s2, d0, d1, d2, tg, th, ts):
    c = lax.axis_index("c")
    s = lax.axis_index("s")
    gtile = c * NS + s
    ebase = gtile * EDGES_PER_TILE

    rows = (rows0, rows1)
    hevs = (hev0, hev1, hev2)
    dstv = (dstv0, dstv1, dstv2)
    gsem = (g0, g1)
    hsem = (h0, h1, h2)
    ssem = (s0, s1, s2)
    dsem = (d0, d1, d2)

    pltpu.sync_copy(src_hbm.at[pl.ds(ebase, EDGES_PER_TILE)], srcv_all)
    pltpu.sync_copy(dtail_hbm.at[gtile], dstv_t)

    def issue_gather(k, br):
        pltpu.async_copy(hv_hbm.at[srcv_all.at[pl.ds(k * CHUNK, CHUNK)]],
                         rows[br], gsem[br])

    def issue_he(k, bh):
        pltpu.async_copy(he_hbm.at[pl.ds(ebase + k * CHUNK, CHUNK)],
                         hevs[bh], hsem[bh])
        pltpu.async_copy(dst3_hbm.at[gtile, k], dstv[bh], dsem[bh])

    def wait_gather(br):
        pltpu.make_async_copy(hv_hbm.at[srcv_all.at[pl.ds(0, CHUNK)]],
                              rows[br], gsem[br]).wait()

    def wait_he(bh):
        pltpu.make_async_copy(he_hbm.at[pl.ds(ebase, CHUNK)],
                              hevs[bh], hsem[bh]).wait()

    def wait_dst(bh):
        pltpu.make_async_copy(dst3_hbm.at[gtile, 0], dstv[bh],
                              dsem[bh]).wait()

    def wait_scatter(bh):
        pltpu.make_async_copy(hevs[bh], acc.at[dstv[bh]], ssem[bh]).wait()

    def compute_scatter(k, br, bh):
        wait_gather(br)
        wait_he(bh)

        def mul(i4, carry):
            for r in range(4):
                i = i4 * 4 + r
                for j in range(8):
                    sl = pl.ds(j * 16, 16)
                    hevs[bh][i, sl] = hevs[bh][i, sl] * rows[br][i, sl]
            return carry

        lax.fori_loop(0, CHUNK // 4, mul, 0)
        wait_dst(bh)
        pltpu.async_copy(hevs[bh], acc.at[dstv[bh]], ssem[bh], add=True)

    def iteration(k, br, bh, bh_next, wait_sc):
        issue_gather(k + 1, 1 - br)
        if wait_sc:
            wait_scatter(bh_next)
        issue_he(k + 1, bh_next)
        compute_scatter(k, br, bh)

    issue_gather(0, 0)
    issue_he(0, 0)
    pltpu.async_copy(hv_hbm.at[srcv_all.at[pl.ds(NCH * CHUNK, TAIL)]],
                     rows_t, tg)
    pltpu.async_copy(he_hbm.at[pl.ds(ebase + NCH * CHUNK, TAIL)], hev_t, th)

    zero = jnp.zeros((16,), jnp.float32)

    def zbody(i, carry):
        for j in range(8):
            hev2[i, pl.ds(j * 16, 16)] = zero
        return carry

    lax.fori_loop(0, CHUNK, zbody, 0)
    base_r = s * ROWS_A

    @pl.when(s < NS - 1)
    def _():
        for k in range(9):
            pltpu.sync_copy(hev2, acc.at[pl.ds(base_r + k * CHUNK, CHUNK)])
        pltpu.sync_copy(hev2.at[pl.ds(0, ROWS_A - 9 * CHUNK)],
                        acc.at[pl.ds(base_r + 9 * CHUNK, ROWS_A - 9 * CHUNK)])

    @pl.when(s == NS - 1)
    def _():
        for k in range(10):
            pltpu.sync_copy(hev2, acc.at[pl.ds(base_r + k * CHUNK, CHUNK)])

    plsc.subcore_barrier()

    iteration(0, 0, 0, 1, False)
    iteration(1, 1, 1, 2, False)

    def steady(t, carry):
        k0 = STEADY0 + t * 6
        for d in range(6):
            k = k0 + d
            iteration(k, (STEADY0 + d) % NR, (STEADY0 + d) % NH,
                      (STEADY0 + d + 1) % NH, True)
        return carry

    lax.fori_loop(0, NSTEADY, steady, 0)
    k_hi = STEADY0 + NSTEADY * 6
    for k in range(k_hi, NCH - 1):
        iteration(k, k % NR, k % NH, (k + 1) % NH, True)
    compute_scatter(NCH - 1, (NCH - 1) % NR, (NCH - 1) % NH)
    for b in range(NH):
        wait_scatter(b)

    pltpu.make_async_copy(hv_hbm.at[srcv_all.at[pl.ds(NCH * CHUNK, TAIL)]],
                          rows_t, tg).wait()
    pltpu.make_async_copy(he_hbm.at[pl.ds(ebase, TAIL)], hev_t, th).wait()

    def mul_t(i, carry):
        for j in range(8):
            sl = pl.ds(j * 16, 16)
            hev_t[i, sl] = hev_t[i, sl] * rows_t[i, sl]
        return carry

    lax.fori_loop(0, TAIL, mul_t, 0)
    pltpu.async_copy(hev_t, acc.at[dstv_t], ts, add=True)
    pltpu.make_async_copy(hev_t, acc.at[dstv_t], ts).wait()

    plsc.subcore_barrier()

    @pl.when(s < NS - 1)
    def _():
        pltpu.sync_copy(acc.at[pl.ds(base_r, ROWS_A)],
                        out_hbm.at[c, pl.ds(base_r, ROWS_A)])

    @pl.when(s == NS - 1)
    def _():
        pltpu.sync_copy(acc.at[pl.ds(base_r, ROWS_LAST)],
                        out_hbm.at[c, pl.ds(base_r, ROWS_LAST)])


@functools.lru_cache(maxsize=1)
def _sc_gather_scatter_fn():
    return pl.kernel(
        _sc_body,
        out_type=jax.ShapeDtypeStruct((NC, N_NODES, EMB), jnp.float32),
        mesh=plsc.VectorSubcoreMesh(core_axis_name="c", subcore_axis_name="s",
                                    num_cores=NC, num_subcores=NS),
        scratch_types=[
            pltpu.VMEM((EDGES_PER_TILE,), jnp.int32),
            pltpu.VMEM((CHUNK,), jnp.int32),
            pltpu.VMEM((CHUNK,), jnp.int32),
            pltpu.VMEM((CHUNK,), jnp.int32),
            pltpu.VMEM((TAIL,), jnp.int32),
            pltpu.VMEM((CHUNK, EMB), jnp.float32),
            pltpu.VMEM((CHUNK, EMB), jnp.float32),
            pltpu.VMEM((CHUNK, EMB), jnp.float32),
            pltpu.VMEM((CHUNK, EMB), jnp.float32),
            pltpu.VMEM((CHUNK, EMB), jnp.float32),
            pltpu.VMEM((TAIL, EMB), jnp.float32),
            pltpu.VMEM((TAIL, EMB), jnp.float32),
            pltpu.VMEM_SHARED((N_NODES, EMB), jnp.float32),
        ] + [pltpu.SemaphoreType.DMA] * 14,
    )


def _sc_gather_scatter(hv, he, src, dst3, dtail):
    return _sc_gather_scatter_fn()(hv, he, src, dst3, dtail)


def kernel(nfeats, edge_index, efeats, embed, pn_W, pn_b, pe_W1, pe_b1, pe_W2,
           pe_b2, co_W, co_b, po_W, po_b, dec_W, dec_b, prelu_a):
    src = edge_index[0].astype(jnp.int32)
    dst = edge_index[1].astype(jnp.int32)
    nfeats = nfeats.astype(jnp.int32)
    dst_r = dst.reshape(NW, EDGES_PER_TILE)
    dst3 = dst_r[:, : NCH * CHUNK].reshape(NW, NCH, CHUNK)
    dtail = dst_r[:, NCH * CHUNK :]

    w1p = jnp.zeros((2, RBF_PAD, EMB), jnp.float32).at[:, :RBF_DIM, :].set(pe_W1)
    w1b = w1p.astype(jnp.bfloat16)
    w2b = pe_W2.astype(jnp.bfloat16)
    pa = jnp.broadcast_to(prelu_a[:, None], (4, EMB))

    he0 = _edge_mlp(efeats, w1b[0], pe_b1[0][None, :], w2b[0], pe_b2[0][None, :])
    hv0 = _hv0(nfeats, embed, pn_W[0], pn_b[0][None, :])

    parts0 = _sc_gather_scatter(hv0, he0, src, dst3, dtail)
    he1 = _edge_mlp(efeats, w1b[1], pe_b1[1][None, :], w2b[1], pe_b2[1][None, :])
    hv1 = _mid(parts0[0], parts0[1], co_W[0], co_b[0][None, :], po_W[0],
               po_b[0][None, :], pn_W[1], pn_b[1][None, :])

    parts1 = _sc_gather_scatter(hv1, he1, src, dst3, dtail)
    out = _decode(parts1[0], parts1[1], co_W[1], co_b[1][None, :], po_W[1],
                  po_b[1][None, :], dec_W, dec_b, pa)
    return out


# speedup vs baseline: 1.3761x; 1.0464x over previous
import functools

import jax
import jax.numpy as jnp
from jax import lax
from jax.experimental import pallas as pl
from jax.experimental.pallas import tpu as pltpu
from jax.experimental.pallas import tpu_sc as plsc

N_NODES = 10000
N_EDGES = 160000
EMB = 128
RBF_DIM = 510
RBF_PAD = 512
CUTOFF = 51.0
INV_GAP = 10.0
LN2 = 0.6931471805599453

NC = 2
NS = 16
NW = NC * NS
EDGES_PER_TILE = N_EDGES // NW
CHUNK = 64
NCH = 78
TAIL = EDGES_PER_TILE - NCH * CHUNK
NR = 2
NH = 3
STEADY0 = 2
NSTEADY = 12
PEEL_HI = 4

ROWS_A = 624
ROWS_LAST = N_NODES - ROWS_A * (NS - 1)


def _ssp(x):
    return jnp.maximum(x, 0.0) + jnp.log(1.0 + jnp.exp(-jnp.abs(x))) - LN2


def _hv0_body(nt_ref, embed_ref, w_ref, b_ref, out_ref):
    nt = nt_ref[...]
    e0 = embed_ref[0, :][None, :]
    e1 = embed_ref[1, :][None, :]
    h0 = jnp.where(nt > 0, e1, e0)
    out_ref[...] = (
        jnp.dot(h0, w_ref[...], preferred_element_type=jnp.float32) + b_ref[...]
    )


GAPC = CUTOFF / (RBF_DIM - 1)


WIN = 128
REACH = 2.0


def _edge_body(ef_ref, w1_ref, b1_ref, w2_ref, b2_ref, he_ref):
    ef = ef_ref[...]
    sq = INV_GAP ** 0.5
    u = jnp.sqrt(jnp.sum(ef * ef, axis=1, keepdims=True)) * sq
    d = u * (1.0 / sq)
    dmin = jnp.min(d)
    dmax = jnp.max(d)

    def _tail(rbf, w1, b1):
        t = _ssp(jnp.dot(rbf, w1, preferred_element_type=jnp.float32)
                 + b1).astype(jnp.bfloat16)
        t = _ssp(jnp.dot(t, w2_ref[...], preferred_element_type=jnp.float32)
                 + b2_ref[...])
        he_ref[...] = t

    narrow = (dmax - dmin) <= (WIN - 9) * GAPC - 2.0 * REACH

    @pl.when(narrow)
    def _():
        s_f = jnp.floor((dmin - REACH) / (8.0 * GAPC)) * 8.0
        start = jnp.clip(s_f.astype(jnp.int32), 0, RBF_PAD - WIN)
        start = pl.multiple_of(start, 8)
        cix = lax.broadcasted_iota(jnp.int32, (1, WIN), 1).astype(jnp.float32)
        centers = (cix + start.astype(jnp.float32)) * (GAPC * sq)
        rbf = jnp.exp(-((u - centers) ** 2)).astype(jnp.bfloat16)
        _tail(rbf, w1_ref[pl.ds(start, WIN), :], b1_ref[...])

    @pl.when(jnp.logical_not(narrow))
    def _():
        cix = lax.broadcasted_iota(jnp.int32, (1, RBF_PAD), 1).astype(
            jnp.float32)
        centers = cix * (GAPC * sq)
        rbf = jnp.exp(-((u - centers) ** 2)).astype(jnp.bfloat16)
        _tail(rbf, w1_ref[...], b1_ref[...])


def _mid_body(p0_ref, p1_ref, cow_ref, cob_ref, pow_ref, pob_ref, pnw_ref,
              pnb_ref, out_ref):
    agg = p0_ref[0] + p1_ref[0]
    t = _ssp(jnp.dot(agg, cow_ref[...], preferred_element_type=jnp.float32)
             + cob_ref[...])
    h = jnp.dot(t, pow_ref[...], preferred_element_type=jnp.float32) + pob_ref[...]
    out_ref[...] = (
        jnp.dot(h, pnw_ref[...], preferred_element_type=jnp.float32) + pnb_ref[...]
    )


def _dec_body(p0_ref, p1_ref, cow_ref, cob_ref, pow_ref, pob_ref, decw_ref,
              decb_ref, pa_ref, out_ref):
    agg = p0_ref[0] + p1_ref[0]
    t = _ssp(jnp.dot(agg, cow_ref[...], preferred_element_type=jnp.float32)
             + cob_ref[...])
    x = jnp.dot(t, pow_ref[...], preferred_element_type=jnp.float32) + pob_ref[...]
    for j in range(4):
        x = jnp.dot(x, decw_ref[j], preferred_element_type=jnp.float32) + decb_ref[j]
        x = jnp.where(x >= 0, x, pa_ref[j] * x)
    out_ref[...] = (
        jnp.dot(x, decw_ref[4], preferred_element_type=jnp.float32) + decb_ref[4]
    )


def _full_spec(a):
    nd = a.ndim
    return pl.BlockSpec(a.shape, lambda i, _nd=nd: (0,) * _nd)


def _hv0(nfeats, embed, w, b):
    bn = 2000
    grid = (N_NODES // bn,)
    return pl.pallas_call(
        _hv0_body,
        grid=grid,
        in_specs=[
            pl.BlockSpec((bn, 1), lambda i: (i, 0)),
            _full_spec(embed),
            _full_spec(w),
            _full_spec(b),
        ],
        out_specs=pl.BlockSpec((bn, EMB), lambda i: (i, 0)),
        out_shape=jax.ShapeDtypeStruct((N_NODES, EMB), jnp.float32),
    )(nfeats, embed, w, b)


def _edge_mlp(efeats, w1p, b1, w2, b2):
    be = 1600
    grid = (N_EDGES // be,)
    return pl.pallas_call(
        _edge_body,
        grid=grid,
        in_specs=[
            pl.BlockSpec((be, 16), lambda i: (i, 0)),
            _full_spec(w1p),
            _full_spec(b1),
            _full_spec(w2),
            _full_spec(b2),
        ],
        out_specs=pl.BlockSpec((be, EMB), lambda i: (i, 0)),
        out_shape=jax.ShapeDtypeStruct((N_EDGES, EMB), jnp.float32),
    )(efeats, w1p, b1, w2, b2)


def _mid(parts, cow, cob, pow_, pob, pnw, pnb):
    bn = 2000
    grid = (N_NODES // bn,)
    return pl.pallas_call(
        _mid_body,
        grid=grid,
        in_specs=[
            pl.BlockSpec((1, bn, EMB), lambda i: (0, i, 0)),
            pl.BlockSpec((1, bn, EMB), lambda i: (1, i, 0)),
            _full_spec(cow), _full_spec(cob),
            _full_spec(pow_), _full_spec(pob),
            _full_spec(pnw), _full_spec(pnb),
        ],
        out_specs=pl.BlockSpec((bn, EMB), lambda i: (i, 0)),
        out_shape=jax.ShapeDtypeStruct((N_NODES, EMB), jnp.float32),
    )(parts, parts, cow, cob, pow_, pob, pnw, pnb)


def _decode(parts, cow, cob, pow_, pob, decw, decb, pa):
    bn = 2000
    grid = (N_NODES // bn,)
    return pl.pallas_call(
        _dec_body,
        grid=grid,
        in_specs=[
            pl.BlockSpec((1, bn, EMB), lambda i: (0, i, 0)),
            pl.BlockSpec((1, bn, EMB), lambda i: (1, i, 0)),
            _full_spec(cow), _full_spec(cob),
            _full_spec(pow_), _full_spec(pob),
            _full_spec(decw), _full_spec(decb), _full_spec(pa),
        ],
        out_specs=pl.BlockSpec((bn, EMB), lambda i: (i, 0)),
        out_shape=jax.ShapeDtypeStruct((N_NODES, EMB), jnp.float32),
    )(parts, parts, cow, cob, pow_, pob, decw, decb, pa)


def _sc_body(hv_hbm, he_hbm, eif_hbm, out_hbm,
             srcv_all, dstv0, dstv1, dstv2, dstv_t,
             rows0, rows1, hev0, hev1, hev2, rows_t, hev_t, acc,
             g0, g1, h0, h1, h2, s0, s1, s2, d0, d1, d2, tg, th, ts):
    c = lax.axis_index("c")
    s = lax.axis_index("s")
    gtile = c * NS + s
    ebase = gtile * EDGES_PER_TILE

    rows = (rows0, rows1)
    hevs = (hev0, hev1, hev2)
    dstv = (dstv0, dstv1, dstv2)
    gsem = (g0, g1)
    hsem = (h0, h1, h2)
    ssem = (s0, s1, s2)
    dsem = (d0, d1, d2)

    pltpu.sync_copy(eif_hbm.at[pl.ds(ebase, EDGES_PER_TILE)], srcv_all)
    pltpu.sync_copy(
        eif_hbm.at[pl.ds(N_EDGES + ebase + NCH * CHUNK, TAIL)], dstv_t)

    def issue_gather(k, br):
        pltpu.async_copy(hv_hbm.at[srcv_all.at[pl.ds(k * CHUNK, CHUNK)]],
                         rows[br], gsem[br])

    def issue_he(k, bh):
        pltpu.async_copy(he_hbm.at[pl.ds(ebase + k * CHUNK, CHUNK)],
                         hevs[bh], hsem[bh])
        pltpu.async_copy(
            eif_hbm.at[pl.ds(N_EDGES + ebase + k * CHUNK, CHUNK)],
            dstv[bh], dsem[bh])

    def wait_gather(br):
        pltpu.make_async_copy(hv_hbm.at[srcv_all.at[pl.ds(0, CHUNK)]],
                              rows[br], gsem[br]).wait()

    def wait_he(bh):
        pltpu.make_async_copy(he_hbm.at[pl.ds(ebase, CHUNK)],
                              hevs[bh], hsem[bh]).wait()

    def wait_dst(bh):
        pltpu.make_async_copy(eif_hbm.at[pl.ds(N_EDGES, CHUNK)], dstv[bh],
                              dsem[bh]).wait()

    def wait_scatter(bh):
        pltpu.make_async_copy(hevs[bh], acc.at[dstv[bh]], ssem[bh]).wait()

    def compute_scatter(k, br, bh):
        wait_gather(br)
        wait_he(bh)

        def mul(i4, carry):
            for r in range(4):
                i = i4 * 4 + r
                for j in range(8):
                    sl = pl.ds(j * 16, 16)
                    hevs[bh][i, sl] = hevs[bh][i, sl] * rows[br][i, sl]
            return carry

        lax.fori_loop(0, CHUNK // 4, mul, 0)
        wait_dst(bh)
        pltpu.async_copy(hevs[bh], acc.at[dstv[bh]], ssem[bh], add=True)

    def iteration(k, br, bh, bh_next, wait_sc):
        issue_gather(k + 1, 1 - br)
        if wait_sc:
            wait_scatter(bh_next)
        issue_he(k + 1, bh_next)
        compute_scatter(k, br, bh)

    issue_gather(0, 0)
    issue_he(0, 0)
    pltpu.async_copy(hv_hbm.at[srcv_all.at[pl.ds(NCH * CHUNK, TAIL)]],
                     rows_t, tg)
    pltpu.async_copy(he_hbm.at[pl.ds(ebase + NCH * CHUNK, TAIL)], hev_t, th)

    zero = jnp.zeros((16,), jnp.float32)

    def zbody(i, carry):
        for j in range(8):
            hev2[i, pl.ds(j * 16, 16)] = zero
        return carry

    lax.fori_loop(0, CHUNK, zbody, 0)
    base_r = s * ROWS_A

    @pl.when(s < NS - 1)
    def _():
        for k in range(9):
            pltpu.sync_copy(hev2, acc.at[pl.ds(base_r + k * CHUNK, CHUNK)])
        pltpu.sync_copy(hev2.at[pl.ds(0, ROWS_A - 9 * CHUNK)],
                        acc.at[pl.ds(base_r + 9 * CHUNK, ROWS_A - 9 * CHUNK)])

    @pl.when(s == NS - 1)
    def _():
        for k in range(10):
            pltpu.sync_copy(hev2, acc.at[pl.ds(base_r + k * CHUNK, CHUNK)])

    plsc.subcore_barrier()

    iteration(0, 0, 0, 1, False)
    iteration(1, 1, 1, 2, False)

    def steady(t, carry):
        k0 = STEADY0 + t * 6
        for d in range(6):
            k = k0 + d
            iteration(k, (STEADY0 + d) % NR, (STEADY0 + d) % NH,
                      (STEADY0 + d + 1) % NH, True)
        return carry

    lax.fori_loop(0, NSTEADY, steady, 0)
    k_hi = STEADY0 + NSTEADY * 6
    for k in range(k_hi, NCH - 1):
        iteration(k, k % NR, k % NH, (k + 1) % NH, True)
    compute_scatter(NCH - 1, (NCH - 1) % NR, (NCH - 1) % NH)
    for b in range(NH):
        wait_scatter(b)

    pltpu.make_async_copy(hv_hbm.at[srcv_all.at[pl.ds(NCH * CHUNK, TAIL)]],
                          rows_t, tg).wait()
    pltpu.make_async_copy(he_hbm.at[pl.ds(ebase, TAIL)], hev_t, th).wait()

    def mul_t(i, carry):
        for j in range(8):
            sl = pl.ds(j * 16, 16)
            hev_t[i, sl] = hev_t[i, sl] * rows_t[i, sl]
        return carry

    lax.fori_loop(0, TAIL, mul_t, 0)
    pltpu.async_copy(hev_t, acc.at[dstv_t], ts, add=True)
    pltpu.make_async_copy(hev_t, acc.at[dstv_t], ts).wait()

    plsc.subcore_barrier()

    @pl.when(s < NS - 1)
    def _():
        pltpu.sync_copy(acc.at[pl.ds(base_r, ROWS_A)],
                        out_hbm.at[c, pl.ds(base_r, ROWS_A)])

    @pl.when(s == NS - 1)
    def _():
        pltpu.sync_copy(acc.at[pl.ds(base_r, ROWS_LAST)],
                        out_hbm.at[c, pl.ds(base_r, ROWS_LAST)])


@functools.lru_cache(maxsize=1)
def _sc_gather_scatter_fn():
    return pl.kernel(
        _sc_body,
        out_type=jax.ShapeDtypeStruct((NC, N_NODES, EMB), jnp.float32),
        mesh=plsc.VectorSubcoreMesh(core_axis_name="c", subcore_axis_name="s",
                                    num_cores=NC, num_subcores=NS),
        scratch_types=[
            pltpu.VMEM((EDGES_PER_TILE,), jnp.int32),
            pltpu.VMEM((CHUNK,), jnp.int32),
            pltpu.VMEM((CHUNK,), jnp.int32),
            pltpu.VMEM((CHUNK,), jnp.int32),
            pltpu.VMEM((TAIL,), jnp.int32),
            pltpu.VMEM((CHUNK, EMB), jnp.float32),
            pltpu.VMEM((CHUNK, EMB), jnp.float32),
            pltpu.VMEM((CHUNK, EMB), jnp.float32),
            pltpu.VMEM((CHUNK, EMB), jnp.float32),
            pltpu.VMEM((CHUNK, EMB), jnp.float32),
            pltpu.VMEM((TAIL, EMB), jnp.float32),
            pltpu.VMEM((TAIL, EMB), jnp.float32),
            pltpu.VMEM_SHARED((N_NODES, EMB), jnp.float32),
        ] + [pltpu.SemaphoreType.DMA] * 14,
    )


def _sc_gather_scatter(hv, he, eif):
    return _sc_gather_scatter_fn()(hv, he, eif)


def kernel(nfeats, edge_index, efeats, embed, pn_W, pn_b, pe_W1, pe_b1, pe_W2,
           pe_b2, co_W, co_b, po_W, po_b, dec_W, dec_b, prelu_a):
    eif = edge_index.astype(jnp.int32).reshape(-1)
    nfeats = nfeats.astype(jnp.int32)

    w1b = (jnp.zeros((2, RBF_PAD, EMB), jnp.bfloat16)
           .at[:, :RBF_DIM, :].set(pe_W1.astype(jnp.bfloat16)))
    w2b = pe_W2.astype(jnp.bfloat16)
    pa = jnp.broadcast_to(prelu_a[:, None], (4, EMB))

    he0 = _edge_mlp(efeats, w1b[0], pe_b1[0][None, :], w2b[0], pe_b2[0][None, :])
    hv0 = _hv0(nfeats, embed, pn_W[0], pn_b[0][None, :])

    parts0 = _sc_gather_scatter(hv0, he0, eif)
    he1 = _edge_mlp(efeats, w1b[1], pe_b1[1][None, :], w2b[1], pe_b2[1][None, :])
    hv1 = _mid(parts0, co_W[0], co_b[0][None, :], po_W[0],
               po_b[0][None, :], pn_W[1], pn_b[1][None, :])

    parts1 = _sc_gather_scatter(hv1, he1, eif)
    out = _decode(parts1, co_W[1], co_b[1][None, :], po_W[1],
                  po_b[1][None, :], dec_W, dec_b, pa)
    return out
